# baseline (speedup 1.0000x reference)
import jax
import jax.numpy as jnp
from jax import lax
from jax.experimental import pallas as pl
from jax.experimental.pallas import tpu as pltpu
from jax.experimental.pallas import tpu_sc as plsc

N = 10000
D = 128
F = 64
NP_ = 10240
NW = 32
CHUNK = 128
SUPER = 3
PER_W = 5376
ITERS = PER_W // (SUPER * CHUNK)
EP = NW * PER_W
ROWS_T = NP_ // 16


def _pre_body(x_ref, wg_ref, asw_ref, adw_ref, xf_ref, w1_ref, b1_ref,
              w2_ref, b2_ref, h_ref, as_ref, ad_ref, h2t_ref):
    h = lax.dot_general(x_ref[...], wg_ref[...], (((1,), (1,)), ((), ())),
                        preferred_element_type=jnp.float32)
    h_ref[...] = h
    as_ref[...] = lax.dot_general(asw_ref[...], h, (((1,), (1,)), ((), ())),
                                  preferred_element_type=jnp.float32)
    ad_ref[...] = lax.dot_general(adw_ref[...], h, (((1,), (1,)), ((), ())),
                                  preferred_element_type=jnp.float32)

    @pl.when(pl.program_id(0) == 0)
    def _():
        w1x = lax.dot_general(w1_ref[...], xf_ref[...], (((1,), (0,)), ((), ())),
                              preferred_element_type=jnp.float32)
        h1t = jnp.maximum(w1x + b1_ref[...], 0.0)
        h2t_ref[...] = lax.dot_general(w2_ref[...], h1t, (((1,), (0,)), ((), ())),
                                       preferred_element_type=jnp.float32) + b2_ref[...]


def _tc_pre(x, W_gat, att_src, att_dst, W1, b1, W2, b2):
    nb = NP_ // 512
    return pl.pallas_call(
        _pre_body,
        grid=(nb,),
        in_specs=[
            pl.BlockSpec((512, D), lambda i: (i, 0)),
            pl.BlockSpec((F, D), lambda i: (0, 0)),
            pl.BlockSpec((1, F), lambda i: (0, 0)),
            pl.BlockSpec((1, F), lambda i: (0, 0)),
            pl.BlockSpec((N, D), lambda i: (0, 0)),
            pl.BlockSpec((F, N), lambda i: (0, 0)),
            pl.BlockSpec((F, 1), lambda i: (0, 0)),
            pl.BlockSpec((F, F), lambda i: (0, 0)),
            pl.BlockSpec((F, 1), lambda i: (0, 0)),
        ],
        out_specs=[
            pl.BlockSpec((512, F), lambda i: (i, 0)),
            pl.BlockSpec((1, 512), lambda i: (0, i)),
            pl.BlockSpec((1, 512), lambda i: (0, i)),
            pl.BlockSpec((F, D), lambda i: (0, 0)),
        ],
        out_shape=[
            jax.ShapeDtypeStruct((NP_, F), jnp.float32),
            jax.ShapeDtypeStruct((1, NP_), jnp.float32),
            jax.ShapeDtypeStruct((1, NP_), jnp.float32),
            jax.ShapeDtypeStruct((F, D), jnp.float32),
        ],
    )(x, W_gat, att_src.reshape(1, F), att_dst.reshape(1, F),
      x, W1, b1.reshape(F, 1), W2, b2.reshape(F, 1))


def _sc_body(srcv_h, dstv_h, asrc_h, adst_h, h_h, z64_h, z1_h,
             outp0_h, outp1_h, den0_h, den1_h,
             asrc_v, adst_v, sidx_v, didx_v, didx2_v, exb_v, rows_v,
             out_sh, den_sh, sem, sem2):
    c = lax.axis_index("c")
    s = lax.axis_index("s")
    wid = c * 16 + s
    base = wid * PER_W
    sb = SUPER * CHUNK

    pltpu.sync_copy(asrc_h, asrc_v)
    pltpu.sync_copy(adst_h, adst_v)
    pltpu.sync_copy(srcv_h.at[pl.ds(base, PER_W)], sidx_v)
    pltpu.sync_copy(dstv_h.at[pl.ds(base, PER_W)], didx_v)
    pltpu.sync_copy(z64_h, out_sh.at[pl.ds(s * ROWS_T, ROWS_T)])
    pltpu.sync_copy(z1_h, den_sh.at[pl.ds(s * ROWS_T, ROWS_T)])
    plsc.subcore_barrier()

    def gather_descs(t, b):
        return [
            pltpu.make_async_copy(
                h_h.at[sidx_v.at[pl.ds(t * sb + k * CHUNK, CHUNK)]],
                rows_v.at[pl.ds(b * sb + k * CHUNK, CHUNK)], sem)
            for k in range(SUPER)
        ]

    def scatter_descs(b):
        ds_ = []
        for k in range(SUPER):
            ds_.append(pltpu.make_async_copy(
                exb_v.at[pl.ds(b * (sb + 16) + k * CHUNK, CHUNK)],
                den_sh.at[didx2_v.at[b * SUPER + k]], sem2))
            ds_.append(pltpu.make_async_copy(
                rows_v.at[pl.ds(b * sb + k * CHUNK, CHUNK)],
                out_sh.at[didx2_v.at[b * SUPER + k]], sem2))
        return ds_

    for d in gather_descs(0, 0):
        d.start()

    def super_blk(t, carry):
        b = lax.rem(t, 2)
        for k in range(SUPER):
            for i in range(8):
                off = t * sb + k * CHUNK + i * 16
                sv = sidx_v[pl.ds(off, 16)]
                dv = didx_v[pl.ds(off, 16)]
                didx2_v[b * SUPER + k, pl.ds(i * 16, 16)] = dv
                a = plsc.load_gather(asrc_v, [sv]) + plsc.load_gather(adst_v, [dv])
                a = jnp.where(a >= 0.0, a, 0.5 * a)
                exb_v[pl.ds(b * (sb + 16) + k * CHUNK + i * 16, 16)] = jnp.exp(a)
        for d in gather_descs(t, b):
            d.wait()

        r0 = b * sb
        e0 = b * (sb + 16)

        def groupf(g, cr):
            exv = exb_v[pl.ds(e0 + g * 16, 16)]
            rg = r0 + g * 16
            for j in range(16):
                scv = exv[j]
                for q in range(4):
                    rows_v[rg + j, pl.ds(q * 16, 16)] = (
                        rows_v[rg + j, pl.ds(q * 16, 16)] * scv)
            return cr
        lax.fori_loop(0, sb // 16, groupf, 0)

        @pl.when(t > 0)
        def _():
            for d in scatter_descs(1 - b):
                d.wait()

        @pl.when(t + 1 < ITERS)
        def _():
            for d in gather_descs(t + 1, 1 - b):
                d.start()

        for d in scatter_descs(b):
            d.start(add=True)
        return carry

    lax.fori_loop(0, ITERS, super_blk, 0)
    for d in scatter_descs((ITERS - 1) % 2):
        d.wait()
    plsc.subcore_barrier()
    rsl = pl.ds(s * ROWS_T, ROWS_T)

    @pl.when(c == 0)
    def _():
        pltpu.sync_copy(out_sh.at[rsl], outp0_h.at[rsl])
        pltpu.sync_copy(den_sh.at[rsl], den0_h.at[rsl])

    @pl.when(c == 1)
    def _():
        pltpu.sync_copy(out_sh.at[rsl], outp1_h.at[rsl])
        pltpu.sync_copy(den_sh.at[rsl], den1_h.at[rsl])


def _sc_call(srcv, dstv, asrc, adst, h, z64, z1):
    mesh = plsc.VectorSubcoreMesh(core_axis_name="c", subcore_axis_name="s")
    return pl.kernel(
        _sc_body,
        out_type=(
            jax.ShapeDtypeStruct((NP_, F), jnp.float32),
            jax.ShapeDtypeStruct((NP_, F), jnp.float32),
            jax.ShapeDtypeStruct((NP_,), jnp.float32),
            jax.ShapeDtypeStruct((NP_,), jnp.float32),
        ),
        mesh=mesh,
        compiler_params=pltpu.CompilerParams(needs_layout_passes=False,
                                             use_tc_tiling_on_sc=False),
        scratch_types=[
            pltpu.VMEM((NP_,), jnp.float32),
            pltpu.VMEM((NP_,), jnp.float32),
            pltpu.VMEM((PER_W,), jnp.int32),
            pltpu.VMEM((PER_W,), jnp.int32),
            pltpu.VMEM((2 * SUPER, CHUNK), jnp.int32),
            pltpu.VMEM((2 * (SUPER * CHUNK + 16),), jnp.float32),
            pltpu.VMEM((2 * SUPER * CHUNK, F), jnp.float32),
            pltpu.VMEM_SHARED((NP_, F), jnp.float32),
            pltpu.VMEM_SHARED((NP_,), jnp.float32),
            pltpu.SemaphoreType.DMA,
            pltpu.SemaphoreType.DMA,
        ],
    )(srcv, dstv, asrc, adst, h, z64, z1)


def _emb_body(o0_ref, o1_ref, d0_ref, d1_ref, bias_ref, h2t_ref,
              emb_ref, xhat_ref):
    o = o0_ref[...] + o1_ref[...]
    dnm = d0_ref[...] + d1_ref[...]
    e = o / (dnm + 1e-16) + bias_ref[...]
    e = jnp.where(e >= 0.0, e, 0.01 * e)
    emb_ref[...] = e
    xhat_ref[...] = lax.dot_general(e, h2t_ref[...], (((1,), (0,)), ((), ())),
                                    preferred_element_type=jnp.float32)


def _tc_emb(outp0, outp1, den0, den1, bias_gat, h2t):
    nb = (N + 511) // 512
    return pl.pallas_call(
        _emb_body,
        grid=(nb,),
        in_specs=[
            pl.BlockSpec((512, F), lambda i: (i, 0)),
            pl.BlockSpec((512, F), lambda i: (i, 0)),
            pl.BlockSpec((512, 1), lambda i: (i, 0)),
            pl.BlockSpec((512, 1), lambda i: (i, 0)),
            pl.BlockSpec((1, F), lambda i: (0, 0)),
            pl.BlockSpec((F, D), lambda i: (0, 0)),
        ],
        out_specs=[
            pl.BlockSpec((512, F), lambda i: (i, 0)),
            pl.BlockSpec((512, D), lambda i: (i, 0)),
        ],
        out_shape=[
            jax.ShapeDtypeStruct((N, F), jnp.float32),
            jax.ShapeDtypeStruct((N, D), jnp.float32),
        ],
    )(outp0, outp1, den0.reshape(NP_, 1), den1.reshape(NP_, 1),
      bias_gat.reshape(1, F), h2t)


def _ahat_body(a_ref, b_ref, o_ref):
    z = lax.dot_general(a_ref[...], b_ref[...], (((1,), (1,)), ((), ())),
                        preferred_element_type=jnp.float32)
    o_ref[...] = 0.5 * jnp.tanh(0.5 * z) + 0.5


def _tc_ahat(emb):
    nbi = (N + 1023) // 1024
    nbj = (N + 4095) // 4096
    return pl.pallas_call(
        _ahat_body,
        grid=(nbi, nbj),
        in_specs=[
            pl.BlockSpec((1024, F), lambda i, j: (i, 0)),
            pl.BlockSpec((4096, F), lambda i, j: (j, 0)),
        ],
        out_specs=pl.BlockSpec((1024, 4096), lambda i, j: (i, j)),
        out_shape=jax.ShapeDtypeStruct((N, N), jnp.float32),
    )(emb, emb)


def kernel(x, edge_index, adj, W_gat, att_src, att_dst, bias_gat, W1, b1, W2, b2):
    e = edge_index.shape[1]
    ei = edge_index.astype(jnp.int32)
    loops = jnp.arange(N, dtype=jnp.int32)
    pad = N + jnp.arange(EP - e - N, dtype=jnp.int32) % (NP_ - N)
    srcv = jnp.concatenate([ei[0], loops, pad])
    dstv = jnp.concatenate([ei[1], loops, pad])

    h, asr, adr, h2t = _tc_pre(x, W_gat, att_src, att_dst, W1, b1, W2, b2)

    z64 = jnp.zeros((ROWS_T, F), jnp.float32)
    z1 = jnp.zeros((ROWS_T,), jnp.float32)
    outp0, outp1, den0, den1 = _sc_call(srcv, dstv, asr.reshape(NP_),
                                        adr.reshape(NP_), h, z64, z1)

    emb, xhat = _tc_emb(outp0, outp1, den0, den1, bias_gat, h2t)
    a_hat = _tc_ahat(emb)
    return (a_hat, xhat)


# speedup vs baseline: 1.1492x; 1.1492x over previous
import jax
import jax.numpy as jnp
from jax import lax
from jax.experimental import pallas as pl
from jax.experimental.pallas import tpu as pltpu
from jax.experimental.pallas import tpu_sc as plsc

N = 10000
D = 128
F = 64
NP_ = 10240
NW = 32
CHUNK = 128
SUPER = 3
PER_W = 5376
ITERS = PER_W // (SUPER * CHUNK)
EP = NW * PER_W
ROWS_T = NP_ // 16


def _pre_body(x_ref, wg_ref, asw_ref, adw_ref, xf_ref, w1_ref, b1_ref,
              w2_ref, b2_ref, h_ref, as_ref, ad_ref, h2t_ref):
    h = lax.dot_general(x_ref[...], wg_ref[...], (((1,), (1,)), ((), ())),
                        preferred_element_type=jnp.float32)
    h_ref[...] = h
    as_ref[...] = lax.dot_general(asw_ref[...], h, (((1,), (1,)), ((), ())),
                                  preferred_element_type=jnp.float32)
    ad_ref[...] = lax.dot_general(adw_ref[...], h, (((1,), (1,)), ((), ())),
                                  preferred_element_type=jnp.float32)

    @pl.when(pl.program_id(0) == 0)
    def _():
        w1x = lax.dot_general(w1_ref[...], xf_ref[...], (((1,), (0,)), ((), ())),
                              preferred_element_type=jnp.float32)
        h1t = jnp.maximum(w1x + b1_ref[...], 0.0)
        h2t_ref[...] = lax.dot_general(w2_ref[...], h1t, (((1,), (0,)), ((), ())),
                                       preferred_element_type=jnp.float32) + b2_ref[...]


def _tc_pre(x, W_gat, att_src, att_dst, W1, b1, W2, b2):
    nb = NP_ // 512
    return pl.pallas_call(
        _pre_body,
        grid=(nb,),
        in_specs=[
            pl.BlockSpec((512, D), lambda i: (i, 0)),
            pl.BlockSpec((F, D), lambda i: (0, 0)),
            pl.BlockSpec((1, F), lambda i: (0, 0)),
            pl.BlockSpec((1, F), lambda i: (0, 0)),
            pl.BlockSpec((N, D), lambda i: (0, 0)),
            pl.BlockSpec((F, N), lambda i: (0, 0)),
            pl.BlockSpec((F, 1), lambda i: (0, 0)),
            pl.BlockSpec((F, F), lambda i: (0, 0)),
            pl.BlockSpec((F, 1), lambda i: (0, 0)),
        ],
        out_specs=[
            pl.BlockSpec((512, F), lambda i: (i, 0)),
            pl.BlockSpec((1, 512), lambda i: (0, i)),
            pl.BlockSpec((1, 512), lambda i: (0, i)),
            pl.BlockSpec((F, D), lambda i: (0, 0)),
        ],
        out_shape=[
            jax.ShapeDtypeStruct((NP_, F), jnp.float32),
            jax.ShapeDtypeStruct((1, NP_), jnp.float32),
            jax.ShapeDtypeStruct((1, NP_), jnp.float32),
            jax.ShapeDtypeStruct((F, D), jnp.float32),
        ],
    )(x, W_gat, att_src.reshape(1, F), att_dst.reshape(1, F),
      x, W1, b1.reshape(F, 1), W2, b2.reshape(F, 1))


def _sc_body(srcv_h, dstv_h, asrc_h, adst_h, h_h, z64_h, z1_h,
             outp0_h, outp1_h, den0_h, den1_h,
             asrc_v, adst_v, sidx_v, didx_v, didx2_v, exb_v, rows_v,
             out_sh, den_sh, sem, sem2):
    c = lax.axis_index("c")
    s = lax.axis_index("s")
    wid = c * 16 + s
    base = wid * PER_W
    sb = SUPER * CHUNK

    pltpu.sync_copy(asrc_h, asrc_v)
    pltpu.sync_copy(adst_h, adst_v)
    pltpu.sync_copy(srcv_h.at[pl.ds(base, PER_W)], sidx_v)
    pltpu.sync_copy(dstv_h.at[pl.ds(base, PER_W)], didx_v)
    pltpu.sync_copy(z64_h, out_sh.at[pl.ds(s * ROWS_T, ROWS_T)])
    pltpu.sync_copy(z1_h, den_sh.at[pl.ds(s * ROWS_T, ROWS_T)])
    plsc.subcore_barrier()

    def gather_descs(t, b):
        return [
            pltpu.make_async_copy(
                h_h.at[sidx_v.at[pl.ds(t * sb + k * CHUNK, CHUNK)]],
                rows_v.at[pl.ds(b * sb + k * CHUNK, CHUNK)], sem)
            for k in range(SUPER)
        ]

    def scatter_descs(b):
        ds_ = []
        for k in range(SUPER):
            ds_.append(pltpu.make_async_copy(
                exb_v.at[pl.ds(b * (sb + 16) + k * CHUNK, CHUNK)],
                den_sh.at[didx2_v.at[b * SUPER + k]], sem2))
            ds_.append(pltpu.make_async_copy(
                rows_v.at[pl.ds(b * sb + k * CHUNK, CHUNK)],
                out_sh.at[didx2_v.at[b * SUPER + k]], sem2))
        return ds_

    for d in gather_descs(0, 0):
        d.start()

    def super_blk(t, carry):
        b = lax.rem(t, 2)
        for k in range(SUPER):
            for i in range(8):
                off = t * sb + k * CHUNK + i * 16
                sv = sidx_v[pl.ds(off, 16)]
                dv = didx_v[pl.ds(off, 16)]
                didx2_v[b * SUPER + k, pl.ds(i * 16, 16)] = dv
                a = plsc.load_gather(asrc_v, [sv]) + plsc.load_gather(adst_v, [dv])
                a = jnp.where(a >= 0.0, a, 0.5 * a)
                exb_v[pl.ds(b * (sb + 16) + k * CHUNK + i * 16, 16)] = jnp.exp(a)
        for d in gather_descs(t, b):
            d.wait()

        r0 = b * sb
        e0 = b * (sb + 16)

        def rowf(r, cr):
            scv = exb_v[pl.ds(e0 + r, 16)][0]
            for q in range(4):
                rows_v[r0 + r, pl.ds(q * 16, 16)] = (
                    rows_v[r0 + r, pl.ds(q * 16, 16)] * scv)
            return cr
        lax.fori_loop(0, sb, rowf, 0, unroll=8)

        @pl.when(t > 0)
        def _():
            for d in scatter_descs(1 - b):
                d.wait()

        @pl.when(t + 1 < ITERS)
        def _():
            for d in gather_descs(t + 1, 1 - b):
                d.start()

        for d in scatter_descs(b):
            d.start(add=True)
        return carry

    lax.fori_loop(0, ITERS, super_blk, 0)
    for d in scatter_descs((ITERS - 1) % 2):
        d.wait()
    plsc.subcore_barrier()
    rsl = pl.ds(s * ROWS_T, ROWS_T)

    @pl.when(c == 0)
    def _():
        pltpu.sync_copy(out_sh.at[rsl], outp0_h.at[rsl])
        pltpu.sync_copy(den_sh.at[rsl], den0_h.at[rsl])

    @pl.when(c == 1)
    def _():
        pltpu.sync_copy(out_sh.at[rsl], outp1_h.at[rsl])
        pltpu.sync_copy(den_sh.at[rsl], den1_h.at[rsl])


def _sc_call(srcv, dstv, asrc, adst, h, z64, z1):
    mesh = plsc.VectorSubcoreMesh(core_axis_name="c", subcore_axis_name="s")
    return pl.kernel(
        _sc_body,
        out_type=(
            jax.ShapeDtypeStruct((NP_, F), jnp.float32),
            jax.ShapeDtypeStruct((NP_, F), jnp.float32),
            jax.ShapeDtypeStruct((NP_,), jnp.float32),
            jax.ShapeDtypeStruct((NP_,), jnp.float32),
        ),
        mesh=mesh,
        compiler_params=pltpu.CompilerParams(needs_layout_passes=False,
                                             use_tc_tiling_on_sc=False),
        scratch_types=[
            pltpu.VMEM((NP_,), jnp.float32),
            pltpu.VMEM((NP_,), jnp.float32),
            pltpu.VMEM((PER_W,), jnp.int32),
            pltpu.VMEM((PER_W,), jnp.int32),
            pltpu.VMEM((2 * SUPER, CHUNK), jnp.int32),
            pltpu.VMEM((2 * (SUPER * CHUNK + 16),), jnp.float32),
            pltpu.VMEM((2 * SUPER * CHUNK, F), jnp.float32),
            pltpu.VMEM_SHARED((NP_, F), jnp.float32),
            pltpu.VMEM_SHARED((NP_,), jnp.float32),
            pltpu.SemaphoreType.DMA,
            pltpu.SemaphoreType.DMA,
        ],
    )(srcv, dstv, asrc, adst, h, z64, z1)


def _emb_body(o0_ref, o1_ref, d0_ref, d1_ref, bias_ref, h2t_ref,
              emb_ref, xhat_ref):
    o = o0_ref[...] + o1_ref[...]
    dnm = d0_ref[...] + d1_ref[...]
    e = o / (dnm + 1e-16) + bias_ref[...]
    e = jnp.where(e >= 0.0, e, 0.01 * e)
    emb_ref[...] = e
    xhat_ref[...] = lax.dot_general(e, h2t_ref[...], (((1,), (0,)), ((), ())),
                                    preferred_element_type=jnp.float32)


def _tc_emb(outp0, outp1, den0, den1, bias_gat, h2t):
    nb = (N + 511) // 512
    return pl.pallas_call(
        _emb_body,
        grid=(nb,),
        in_specs=[
            pl.BlockSpec((512, F), lambda i: (i, 0)),
            pl.BlockSpec((512, F), lambda i: (i, 0)),
            pl.BlockSpec((512, 1), lambda i: (i, 0)),
            pl.BlockSpec((512, 1), lambda i: (i, 0)),
            pl.BlockSpec((1, F), lambda i: (0, 0)),
            pl.BlockSpec((F, D), lambda i: (0, 0)),
        ],
        out_specs=[
            pl.BlockSpec((512, F), lambda i: (i, 0)),
            pl.BlockSpec((512, D), lambda i: (i, 0)),
        ],
        out_shape=[
            jax.ShapeDtypeStruct((N, F), jnp.float32),
            jax.ShapeDtypeStruct((N, D), jnp.float32),
        ],
    )(outp0, outp1, den0.reshape(NP_, 1), den1.reshape(NP_, 1),
      bias_gat.reshape(1, F), h2t)


def _ahat_body(a_ref, b_ref, o_ref):
    z = lax.dot_general(a_ref[...], b_ref[...], (((1,), (1,)), ((), ())),
                        preferred_element_type=jnp.float32)
    o_ref[...] = 0.5 * jnp.tanh(0.5 * z) + 0.5


def _tc_ahat(emb):
    nbi = (N + 1023) // 1024
    nbj = (N + 4095) // 4096
    return pl.pallas_call(
        _ahat_body,
        grid=(nbi, nbj),
        in_specs=[
            pl.BlockSpec((1024, F), lambda i, j: (i, 0)),
            pl.BlockSpec((4096, F), lambda i, j: (j, 0)),
        ],
        out_specs=pl.BlockSpec((1024, 4096), lambda i, j: (i, j)),
        out_shape=jax.ShapeDtypeStruct((N, N), jnp.float32),
    )(emb, emb)


def kernel(x, edge_index, adj, W_gat, att_src, att_dst, bias_gat, W1, b1, W2, b2):
    e = edge_index.shape[1]
    ei = edge_index.astype(jnp.int32)
    loops = jnp.arange(N, dtype=jnp.int32)
    pad = N + jnp.arange(EP - e - N, dtype=jnp.int32) % (NP_ - N)
    srcv = jnp.concatenate([ei[0], loops, pad])
    dstv = jnp.concatenate([ei[1], loops, pad])

    h, asr, adr, h2t = _tc_pre(x, W_gat, att_src, att_dst, W1, b1, W2, b2)

    z64 = jnp.zeros((ROWS_T, F), jnp.float32)
    z1 = jnp.zeros((ROWS_T,), jnp.float32)
    outp0, outp1, den0, den1 = _sc_call(srcv, dstv, asr.reshape(NP_),
                                        adr.reshape(NP_), h, z64, z1)

    emb, xhat = _tc_emb(outp0, outp1, den0, den1, bias_gat, h2t)
    a_hat = _tc_ahat(emb)
    return (a_hat, xhat)


# speedup vs baseline: 1.1950x; 1.0399x over previous
import jax
import jax.numpy as jnp
from jax import lax
from jax.experimental import pallas as pl
from jax.experimental.pallas import tpu as pltpu
from jax.experimental.pallas import tpu_sc as plsc

N = 10000
D = 128
F = 64
NP_ = 10240
NW = 32
CHUNK = 128
SUPER = 3
PER_W = 5376
ITERS = PER_W // (SUPER * CHUNK)
EP = NW * PER_W
ROWS_T = NP_ // 16


def _pre_body(x_ref, wg_ref, asw_ref, adw_ref, xf_ref, w1_ref, b1_ref,
              w2_ref, b2_ref, h_ref, as_ref, ad_ref, h2t_ref):
    h = lax.dot_general(x_ref[...], wg_ref[...], (((1,), (1,)), ((), ())),
                        preferred_element_type=jnp.float32)
    h_ref[...] = h
    as_ref[...] = lax.dot_general(asw_ref[...], h, (((1,), (1,)), ((), ())),
                                  preferred_element_type=jnp.float32)
    ad_ref[...] = lax.dot_general(adw_ref[...], h, (((1,), (1,)), ((), ())),
                                  preferred_element_type=jnp.float32)

    @pl.when(pl.program_id(0) == 0)
    def _():
        w1x = lax.dot_general(w1_ref[...], xf_ref[...], (((1,), (0,)), ((), ())),
                              preferred_element_type=jnp.float32)
        h1t = jnp.maximum(w1x + b1_ref[...], 0.0)
        h2t_ref[...] = lax.dot_general(w2_ref[...], h1t, (((1,), (0,)), ((), ())),
                                       preferred_element_type=jnp.float32) + b2_ref[...]


def _tc_pre(x, W_gat, att_src, att_dst, W1, b1, W2, b2):
    nb = NP_ // 512
    return pl.pallas_call(
        _pre_body,
        grid=(nb,),
        in_specs=[
            pl.BlockSpec((512, D), lambda i: (i, 0)),
            pl.BlockSpec((F, D), lambda i: (0, 0)),
            pl.BlockSpec((1, F), lambda i: (0, 0)),
            pl.BlockSpec((1, F), lambda i: (0, 0)),
            pl.BlockSpec((N, D), lambda i: (0, 0)),
            pl.BlockSpec((F, N), lambda i: (0, 0)),
            pl.BlockSpec((F, 1), lambda i: (0, 0)),
            pl.BlockSpec((F, F), lambda i: (0, 0)),
            pl.BlockSpec((F, 1), lambda i: (0, 0)),
        ],
        out_specs=[
            pl.BlockSpec((512, F), lambda i: (i, 0)),
            pl.BlockSpec((1, 512), lambda i: (0, i)),
            pl.BlockSpec((1, 512), lambda i: (0, i)),
            pl.BlockSpec((F, D), lambda i: (0, 0)),
        ],
        out_shape=[
            jax.ShapeDtypeStruct((NP_, F), jnp.float32),
            jax.ShapeDtypeStruct((1, NP_), jnp.float32),
            jax.ShapeDtypeStruct((1, NP_), jnp.float32),
            jax.ShapeDtypeStruct((F, D), jnp.float32),
        ],
    )(x, W_gat, att_src.reshape(1, F), att_dst.reshape(1, F),
      x, W1, b1.reshape(F, 1), W2, b2.reshape(F, 1))


def _sc_body(srcv_h, dstv_h, asrc_h, adst_h, h_h, z64_h, z1_h,
             outp0_h, outp1_h, den0_h, den1_h,
             asrc_v, adst_v, sidx_v, didx_v, didx2_v, exb_v, rows_v,
             out_sh, den_sh, sem, sem2):
    c = lax.axis_index("c")
    s = lax.axis_index("s")
    wid = c * 16 + s
    base = wid * PER_W
    sb = SUPER * CHUNK

    pltpu.sync_copy(asrc_h, asrc_v)
    pltpu.sync_copy(adst_h, adst_v)
    pltpu.sync_copy(srcv_h.at[pl.ds(base, PER_W)], sidx_v)
    pltpu.sync_copy(dstv_h.at[pl.ds(base, PER_W)], didx_v)
    pltpu.sync_copy(z64_h, out_sh.at[pl.ds(s * ROWS_T, ROWS_T)])
    pltpu.sync_copy(z1_h, den_sh.at[pl.ds(s * ROWS_T, ROWS_T)])
    plsc.subcore_barrier()

    def gather_descs(t, b):
        return [
            pltpu.make_async_copy(
                h_h.at[sidx_v.at[pl.ds(t * sb + k * CHUNK, CHUNK)]],
                rows_v.at[pl.ds(b * sb + k * CHUNK, CHUNK)], sem)
            for k in range(SUPER)
        ]

    def scatter_descs(b):
        ds_ = []
        for k in range(SUPER):
            ds_.append(pltpu.make_async_copy(
                exb_v.at[pl.ds(b * (sb + 16) + k * CHUNK, CHUNK)],
                den_sh.at[didx2_v.at[b * SUPER + k]], sem2))
            ds_.append(pltpu.make_async_copy(
                rows_v.at[pl.ds(b * sb + k * CHUNK, CHUNK)],
                out_sh.at[didx2_v.at[b * SUPER + k]], sem2))
        return ds_

    for d in gather_descs(0, 0):
        d.start()

    def super_blk(t, carry):
        b = lax.rem(t, 2)
        for k in range(SUPER):
            for i in range(8):
                off = t * sb + k * CHUNK + i * 16
                sv = sidx_v[pl.ds(off, 16)]
                dv = didx_v[pl.ds(off, 16)]
                didx2_v[b * SUPER + k, pl.ds(i * 16, 16)] = dv
                a = plsc.load_gather(asrc_v, [sv]) + plsc.load_gather(adst_v, [dv])
                a = jnp.where(a >= 0.0, a, 0.5 * a)
                exb_v[pl.ds(b * (sb + 16) + k * CHUNK + i * 16, 16)] = jnp.exp(a)
        for d in gather_descs(t, b):
            d.wait()
        @pl.when(t > 0)
        def _():
            for d in scatter_descs(1 - b):
                d.wait()

        @pl.when(t + 1 < ITERS)
        def _():
            for d in gather_descs(t + 1, 1 - b):
                d.start()

        r0 = b * sb
        e0 = b * (sb + 16)

        def rowf(r, cr):
            scv = exb_v[pl.ds(e0 + r, 16)][0]
            for q in range(4):
                rows_v[r0 + r, pl.ds(q * 16, 16)] = (
                    rows_v[r0 + r, pl.ds(q * 16, 16)] * scv)
            return cr
        lax.fori_loop(0, sb, rowf, 0, unroll=8)

        for d in scatter_descs(b):
            d.start(add=True)
        return carry

    lax.fori_loop(0, ITERS, super_blk, 0)
    for d in scatter_descs((ITERS - 1) % 2):
        d.wait()
    plsc.subcore_barrier()
    rsl = pl.ds(s * ROWS_T, ROWS_T)

    @pl.when(c == 0)
    def _():
        pltpu.sync_copy(out_sh.at[rsl], outp0_h.at[rsl])
        pltpu.sync_copy(den_sh.at[rsl], den0_h.at[rsl])

    @pl.when(c == 1)
    def _():
        pltpu.sync_copy(out_sh.at[rsl], outp1_h.at[rsl])
        pltpu.sync_copy(den_sh.at[rsl], den1_h.at[rsl])


def _sc_call(srcv, dstv, asrc, adst, h, z64, z1):
    mesh = plsc.VectorSubcoreMesh(core_axis_name="c", subcore_axis_name="s")
    return pl.kernel(
        _sc_body,
        out_type=(
            jax.ShapeDtypeStruct((NP_, F), jnp.float32),
            jax.ShapeDtypeStruct((NP_, F), jnp.float32),
            jax.ShapeDtypeStruct((NP_,), jnp.float32),
            jax.ShapeDtypeStruct((NP_,), jnp.float32),
        ),
        mesh=mesh,
        compiler_params=pltpu.CompilerParams(needs_layout_passes=False,
                                             use_tc_tiling_on_sc=False),
        scratch_types=[
            pltpu.VMEM((NP_,), jnp.float32),
            pltpu.VMEM((NP_,), jnp.float32),
            pltpu.VMEM((PER_W,), jnp.int32),
            pltpu.VMEM((PER_W,), jnp.int32),
            pltpu.VMEM((2 * SUPER, CHUNK), jnp.int32),
            pltpu.VMEM((2 * (SUPER * CHUNK + 16),), jnp.float32),
            pltpu.VMEM((2 * SUPER * CHUNK, F), jnp.float32),
            pltpu.VMEM_SHARED((NP_, F), jnp.float32),
            pltpu.VMEM_SHARED((NP_,), jnp.float32),
            pltpu.SemaphoreType.DMA,
            pltpu.SemaphoreType.DMA,
        ],
    )(srcv, dstv, asrc, adst, h, z64, z1)


def _emb_body(o0_ref, o1_ref, d0_ref, d1_ref, bias_ref, h2t_ref,
              emb_ref, xhat_ref):
    o = o0_ref[...] + o1_ref[...]
    dnm = d0_ref[...] + d1_ref[...]
    e = o / (dnm + 1e-16) + bias_ref[...]
    e = jnp.where(e >= 0.0, e, 0.01 * e)
    emb_ref[...] = e
    xhat_ref[...] = lax.dot_general(e, h2t_ref[...], (((1,), (0,)), ((), ())),
                                    preferred_element_type=jnp.float32)


def _tc_emb(outp0, outp1, den0, den1, bias_gat, h2t):
    nb = (N + 511) // 512
    return pl.pallas_call(
        _emb_body,
        grid=(nb,),
        in_specs=[
            pl.BlockSpec((512, F), lambda i: (i, 0)),
            pl.BlockSpec((512, F), lambda i: (i, 0)),
            pl.BlockSpec((512, 1), lambda i: (i, 0)),
            pl.BlockSpec((512, 1), lambda i: (i, 0)),
            pl.BlockSpec((1, F), lambda i: (0, 0)),
            pl.BlockSpec((F, D), lambda i: (0, 0)),
        ],
        out_specs=[
            pl.BlockSpec((512, F), lambda i: (i, 0)),
            pl.BlockSpec((512, D), lambda i: (i, 0)),
        ],
        out_shape=[
            jax.ShapeDtypeStruct((N, F), jnp.float32),
            jax.ShapeDtypeStruct((N, D), jnp.float32),
        ],
    )(outp0, outp1, den0.reshape(NP_, 1), den1.reshape(NP_, 1),
      bias_gat.reshape(1, F), h2t)


def _ahat_body(a_ref, b_ref, o_ref):
    z = lax.dot_general(a_ref[...], b_ref[...], (((1,), (1,)), ((), ())),
                        preferred_element_type=jnp.float32)
    o_ref[...] = 0.5 * jnp.tanh(0.5 * z) + 0.5


def _tc_ahat(emb):
    nbi = (N + 1023) // 1024
    nbj = (N + 4095) // 4096
    return pl.pallas_call(
        _ahat_body,
        grid=(nbi, nbj),
        in_specs=[
            pl.BlockSpec((1024, F), lambda i, j: (i, 0)),
            pl.BlockSpec((4096, F), lambda i, j: (j, 0)),
        ],
        out_specs=pl.BlockSpec((1024, 4096), lambda i, j: (i, j)),
        out_shape=jax.ShapeDtypeStruct((N, N), jnp.float32),
    )(emb, emb)


def kernel(x, edge_index, adj, W_gat, att_src, att_dst, bias_gat, W1, b1, W2, b2):
    e = edge_index.shape[1]
    ei = edge_index.astype(jnp.int32)
    loops = jnp.arange(N, dtype=jnp.int32)
    pad = N + jnp.arange(EP - e - N, dtype=jnp.int32) % (NP_ - N)
    srcv = jnp.concatenate([ei[0], loops, pad])
    dstv = jnp.concatenate([ei[1], loops, pad])

    h, asr, adr, h2t = _tc_pre(x, W_gat, att_src, att_dst, W1, b1, W2, b2)

    z64 = jnp.zeros((ROWS_T, F), jnp.float32)
    z1 = jnp.zeros((ROWS_T,), jnp.float32)
    outp0, outp1, den0, den1 = _sc_call(srcv, dstv, asr.reshape(NP_),
                                        adr.reshape(NP_), h, z64, z1)

    emb, xhat = _tc_emb(outp0, outp1, den0, den1, bias_gat, h2t)
    a_hat = _tc_ahat(emb)
    return (a_hat, xhat)


# speedup vs baseline: 1.2705x; 1.0632x over previous
import jax
import jax.numpy as jnp
from jax import lax
from jax.experimental import pallas as pl
from jax.experimental.pallas import tpu as pltpu
from jax.experimental.pallas import tpu_sc as plsc

N = 10000
D = 128
F = 64
NP_ = 10240
NW = 32
CHUNK = 128
SUPER = 3
PER_W = 5376
ITERS = PER_W // (SUPER * CHUNK)
EP = NW * PER_W
ROWS_T = NP_ // 16


def _pre_body(x_ref, wg_ref, asw_ref, adw_ref, xf_ref, w1_ref, b1_ref,
              w2_ref, b2_ref, h_ref, as_ref, ad_ref, h2t_ref):
    h = lax.dot_general(x_ref[...], wg_ref[...], (((1,), (1,)), ((), ())),
                        preferred_element_type=jnp.float32)
    h_ref[...] = h
    as_ref[...] = lax.dot_general(asw_ref[...], h, (((1,), (1,)), ((), ())),
                                  preferred_element_type=jnp.float32)
    ad_ref[...] = lax.dot_general(adw_ref[...], h, (((1,), (1,)), ((), ())),
                                  preferred_element_type=jnp.float32)

    @pl.when(pl.program_id(0) == 0)
    def _():
        w1x = lax.dot_general(w1_ref[...], xf_ref[...], (((1,), (0,)), ((), ())),
                              preferred_element_type=jnp.float32)
        h1t = jnp.maximum(w1x + b1_ref[...], 0.0)
        h2t_ref[...] = lax.dot_general(w2_ref[...], h1t, (((1,), (0,)), ((), ())),
                                       preferred_element_type=jnp.float32) + b2_ref[...]


def _tc_pre(x, W_gat, att_src, att_dst, W1, b1, W2, b2):
    nb = NP_ // 512
    return pl.pallas_call(
        _pre_body,
        grid=(nb,),
        in_specs=[
            pl.BlockSpec((512, D), lambda i: (i, 0)),
            pl.BlockSpec((F, D), lambda i: (0, 0)),
            pl.BlockSpec((1, F), lambda i: (0, 0)),
            pl.BlockSpec((1, F), lambda i: (0, 0)),
            pl.BlockSpec((N, D), lambda i: (0, 0)),
            pl.BlockSpec((F, N), lambda i: (0, 0)),
            pl.BlockSpec((F, 1), lambda i: (0, 0)),
            pl.BlockSpec((F, F), lambda i: (0, 0)),
            pl.BlockSpec((F, 1), lambda i: (0, 0)),
        ],
        out_specs=[
            pl.BlockSpec((512, F), lambda i: (i, 0)),
            pl.BlockSpec((1, 512), lambda i: (0, i)),
            pl.BlockSpec((1, 512), lambda i: (0, i)),
            pl.BlockSpec((F, D), lambda i: (0, 0)),
        ],
        out_shape=[
            jax.ShapeDtypeStruct((NP_, F), jnp.float32),
            jax.ShapeDtypeStruct((1, NP_), jnp.float32),
            jax.ShapeDtypeStruct((1, NP_), jnp.float32),
            jax.ShapeDtypeStruct((F, D), jnp.float32),
        ],
    )(x, W_gat, att_src.reshape(1, F), att_dst.reshape(1, F),
      x, W1, b1.reshape(F, 1), W2, b2.reshape(F, 1))


def _sc_body(srcv_h, dstv_h, asrc_h, adst_h, h_h, z64_h, z1_h,
             outp0_h, outp1_h, den0_h, den1_h,
             asrc_v, adst_v, sidx_v, didx_v, didx2_v, exb_v, rows_v,
             out_sh, den_sh, sem, sem2):
    c = lax.axis_index("c")
    s = lax.axis_index("s")
    wid = c * 16 + s
    base = wid * PER_W
    sb = SUPER * CHUNK

    pltpu.sync_copy(asrc_h, asrc_v)
    pltpu.sync_copy(adst_h, adst_v)
    pltpu.sync_copy(srcv_h.at[pl.ds(base, PER_W)], sidx_v)
    pltpu.sync_copy(dstv_h.at[pl.ds(base, PER_W)], didx_v)
    pltpu.sync_copy(z64_h, out_sh.at[pl.ds(s * ROWS_T, ROWS_T)])
    pltpu.sync_copy(z1_h, den_sh.at[pl.ds(s * ROWS_T, ROWS_T)])
    plsc.subcore_barrier()

    def gather_descs(t, b):
        return [
            pltpu.make_async_copy(
                h_h.at[sidx_v.at[pl.ds(t * sb + k * CHUNK, CHUNK)]],
                rows_v.at[pl.ds(b * sb + k * CHUNK, CHUNK)], sem)
            for k in range(SUPER)
        ]

    def scatter_descs(b):
        ds_ = []
        for k in range(SUPER):
            ds_.append(pltpu.make_async_copy(
                exb_v.at[pl.ds(b * (sb + 16) + k * CHUNK, CHUNK)],
                den_sh.at[didx2_v.at[b * SUPER + k]], sem2))
            ds_.append(pltpu.make_async_copy(
                rows_v.at[pl.ds(b * sb + k * CHUNK, CHUNK)],
                out_sh.at[didx2_v.at[b * SUPER + k]], sem2))
        return ds_

    for d in gather_descs(0, 0):
        d.start()

    def super_blk(t, carry):
        b = lax.rem(t, 2)
        for k in range(SUPER):
            for i in range(8):
                off = t * sb + k * CHUNK + i * 16
                sv = sidx_v[pl.ds(off, 16)]
                dv = didx_v[pl.ds(off, 16)]
                didx2_v[b * SUPER + k, pl.ds(i * 16, 16)] = dv
                a = plsc.load_gather(asrc_v, [sv]) + plsc.load_gather(adst_v, [dv])
                a = jnp.where(a >= 0.0, a, 0.5 * a)
                exb_v[pl.ds(b * (sb + 16) + k * CHUNK + i * 16, 16)] = jnp.exp(a)
        for d in gather_descs(t, b):
            d.wait()
        @pl.when(t > 0)
        def _():
            for d in scatter_descs(1 - b):
                d.wait()

        @pl.when(t + 1 < ITERS)
        def _():
            for d in gather_descs(t + 1, 1 - b):
                d.start()

        r0 = b * sb
        e0 = b * (sb + 16)

        @plsc.parallel_loop(0, sb, unroll=8)
        def _(r):
            scv = exb_v[pl.ds(e0 + r, 16)][0]
            for q in range(4):
                rows_v[r0 + r, pl.ds(q * 16, 16)] = (
                    rows_v[r0 + r, pl.ds(q * 16, 16)] * scv)

        for d in scatter_descs(b):
            d.start(add=True)
        return carry

    lax.fori_loop(0, ITERS, super_blk, 0)
    for d in scatter_descs((ITERS - 1) % 2):
        d.wait()
    plsc.subcore_barrier()
    rsl = pl.ds(s * ROWS_T, ROWS_T)

    @pl.when(c == 0)
    def _():
        pltpu.sync_copy(out_sh.at[rsl], outp0_h.at[rsl])
        pltpu.sync_copy(den_sh.at[rsl], den0_h.at[rsl])

    @pl.when(c == 1)
    def _():
        pltpu.sync_copy(out_sh.at[rsl], outp1_h.at[rsl])
        pltpu.sync_copy(den_sh.at[rsl], den1_h.at[rsl])


def _sc_call(srcv, dstv, asrc, adst, h, z64, z1):
    mesh = plsc.VectorSubcoreMesh(core_axis_name="c", subcore_axis_name="s")
    return pl.kernel(
        _sc_body,
        out_type=(
            jax.ShapeDtypeStruct((NP_, F), jnp.float32),
            jax.ShapeDtypeStruct((NP_, F), jnp.float32),
            jax.ShapeDtypeStruct((NP_,), jnp.float32),
            jax.ShapeDtypeStruct((NP_,), jnp.float32),
        ),
        mesh=mesh,
        compiler_params=pltpu.CompilerParams(needs_layout_passes=False,
                                             use_tc_tiling_on_sc=False),
        scratch_types=[
            pltpu.VMEM((NP_,), jnp.float32),
            pltpu.VMEM((NP_,), jnp.float32),
            pltpu.VMEM((PER_W,), jnp.int32),
            pltpu.VMEM((PER_W,), jnp.int32),
            pltpu.VMEM((2 * SUPER, CHUNK), jnp.int32),
            pltpu.VMEM((2 * (SUPER * CHUNK + 16),), jnp.float32),
            pltpu.VMEM((2 * SUPER * CHUNK, F), jnp.float32),
            pltpu.VMEM_SHARED((NP_, F), jnp.float32),
            pltpu.VMEM_SHARED((NP_,), jnp.float32),
            pltpu.SemaphoreType.DMA,
            pltpu.SemaphoreType.DMA,
        ],
    )(srcv, dstv, asrc, adst, h, z64, z1)


def _emb_body(o0_ref, o1_ref, d0_ref, d1_ref, bias_ref, h2t_ref,
              emb_ref, xhat_ref):
    o = o0_ref[...] + o1_ref[...]
    dnm = d0_ref[...] + d1_ref[...]
    e = o / (dnm + 1e-16) + bias_ref[...]
    e = jnp.where(e >= 0.0, e, 0.01 * e)
    emb_ref[...] = e
    xhat_ref[...] = lax.dot_general(e, h2t_ref[...], (((1,), (0,)), ((), ())),
                                    preferred_element_type=jnp.float32)


def _tc_emb(outp0, outp1, den0, den1, bias_gat, h2t):
    nb = (N + 511) // 512
    return pl.pallas_call(
        _emb_body,
        grid=(nb,),
        in_specs=[
            pl.BlockSpec((512, F), lambda i: (i, 0)),
            pl.BlockSpec((512, F), lambda i: (i, 0)),
            pl.BlockSpec((512, 1), lambda i: (i, 0)),
            pl.BlockSpec((512, 1), lambda i: (i, 0)),
            pl.BlockSpec((1, F), lambda i: (0, 0)),
            pl.BlockSpec((F, D), lambda i: (0, 0)),
        ],
        out_specs=[
            pl.BlockSpec((512, F), lambda i: (i, 0)),
            pl.BlockSpec((512, D), lambda i: (i, 0)),
        ],
        out_shape=[
            jax.ShapeDtypeStruct((N, F), jnp.float32),
            jax.ShapeDtypeStruct((N, D), jnp.float32),
        ],
    )(outp0, outp1, den0.reshape(NP_, 1), den1.reshape(NP_, 1),
      bias_gat.reshape(1, F), h2t)


def _ahat_body(a_ref, b_ref, o_ref):
    z = lax.dot_general(a_ref[...], b_ref[...], (((1,), (1,)), ((), ())),
                        preferred_element_type=jnp.float32)
    o_ref[...] = 0.5 * jnp.tanh(0.5 * z) + 0.5


def _tc_ahat(emb):
    nbi = (N + 1023) // 1024
    nbj = (N + 4095) // 4096
    return pl.pallas_call(
        _ahat_body,
        grid=(nbi, nbj),
        in_specs=[
            pl.BlockSpec((1024, F), lambda i, j: (i, 0)),
            pl.BlockSpec((4096, F), lambda i, j: (j, 0)),
        ],
        out_specs=pl.BlockSpec((1024, 4096), lambda i, j: (i, j)),
        out_shape=jax.ShapeDtypeStruct((N, N), jnp.float32),
    )(emb, emb)


def kernel(x, edge_index, adj, W_gat, att_src, att_dst, bias_gat, W1, b1, W2, b2):
    e = edge_index.shape[1]
    ei = edge_index.astype(jnp.int32)
    loops = jnp.arange(N, dtype=jnp.int32)
    pad = N + jnp.arange(EP - e - N, dtype=jnp.int32) % (NP_ - N)
    srcv = jnp.concatenate([ei[0], loops, pad])
    dstv = jnp.concatenate([ei[1], loops, pad])

    h, asr, adr, h2t = _tc_pre(x, W_gat, att_src, att_dst, W1, b1, W2, b2)

    z64 = jnp.zeros((ROWS_T, F), jnp.float32)
    z1 = jnp.zeros((ROWS_T,), jnp.float32)
    outp0, outp1, den0, den1 = _sc_call(srcv, dstv, asr.reshape(NP_),
                                        adr.reshape(NP_), h, z64, z1)

    emb, xhat = _tc_emb(outp0, outp1, den0, den1, bias_gat, h2t)
    a_hat = _tc_ahat(emb)
    return (a_hat, xhat)


# speedup vs baseline: 1.2710x; 1.0004x over previous
import jax
import jax.numpy as jnp
from jax import lax
from jax.experimental import pallas as pl
from jax.experimental.pallas import tpu as pltpu
from jax.experimental.pallas import tpu_sc as plsc

N = 10000
D = 128
F = 64
NP_ = 10240
NW = 32
CHUNK = 128
SUPER = 3
PER_W = 5376
ITERS = PER_W // (SUPER * CHUNK)
EP = NW * PER_W
ROWS_T = NP_ // 16


def _pre_body(x_ref, wg_ref, asw_ref, adw_ref, xf_ref, w1_ref, b1_ref,
              w2_ref, b2_ref, h_ref, as_ref, ad_ref, h2t_ref):
    h = lax.dot_general(x_ref[...], wg_ref[...], (((1,), (1,)), ((), ())),
                        preferred_element_type=jnp.float32)
    h_ref[...] = h
    as_ref[...] = lax.dot_general(asw_ref[...], h, (((1,), (1,)), ((), ())),
                                  preferred_element_type=jnp.float32)
    ad_ref[...] = lax.dot_general(adw_ref[...], h, (((1,), (1,)), ((), ())),
                                  preferred_element_type=jnp.float32)

    @pl.when(pl.program_id(0) == 0)
    def _():
        w1x = lax.dot_general(w1_ref[...], xf_ref[...], (((1,), (0,)), ((), ())),
                              preferred_element_type=jnp.float32)
        h1t = jnp.maximum(w1x + b1_ref[...], 0.0)
        h2t_ref[...] = lax.dot_general(w2_ref[...], h1t, (((1,), (0,)), ((), ())),
                                       preferred_element_type=jnp.float32) + b2_ref[...]


def _tc_pre(x, W_gat, att_src, att_dst, W1, b1, W2, b2):
    nb = NP_ // 512
    return pl.pallas_call(
        _pre_body,
        grid=(nb,),
        in_specs=[
            pl.BlockSpec((512, D), lambda i: (i, 0)),
            pl.BlockSpec((F, D), lambda i: (0, 0)),
            pl.BlockSpec((1, F), lambda i: (0, 0)),
            pl.BlockSpec((1, F), lambda i: (0, 0)),
            pl.BlockSpec((N, D), lambda i: (0, 0)),
            pl.BlockSpec((F, N), lambda i: (0, 0)),
            pl.BlockSpec((F, 1), lambda i: (0, 0)),
            pl.BlockSpec((F, F), lambda i: (0, 0)),
            pl.BlockSpec((F, 1), lambda i: (0, 0)),
        ],
        out_specs=[
            pl.BlockSpec((512, F), lambda i: (i, 0)),
            pl.BlockSpec((1, 512), lambda i: (0, i)),
            pl.BlockSpec((1, 512), lambda i: (0, i)),
            pl.BlockSpec((F, D), lambda i: (0, 0)),
        ],
        out_shape=[
            jax.ShapeDtypeStruct((NP_, F), jnp.float32),
            jax.ShapeDtypeStruct((1, NP_), jnp.float32),
            jax.ShapeDtypeStruct((1, NP_), jnp.float32),
            jax.ShapeDtypeStruct((F, D), jnp.float32),
        ],
    )(x, W_gat, att_src.reshape(1, F), att_dst.reshape(1, F),
      x, W1, b1.reshape(F, 1), W2, b2.reshape(F, 1))


def _sc_body(srcv_h, dstv_h, asrc_h, adst_h, h_h, z64_h, z1_h,
             outp0_h, outp1_h, den0_h, den1_h,
             asrc_v, adst_v, sidx_v, didx_v, didx2_v, exb_v, rows_v,
             out_sh, den_sh, sem, sem2):
    c = lax.axis_index("c")
    s = lax.axis_index("s")
    wid = c * 16 + s
    base = wid * PER_W
    sb = SUPER * CHUNK

    pltpu.sync_copy(asrc_h, asrc_v)
    pltpu.sync_copy(adst_h, adst_v)
    pltpu.sync_copy(srcv_h.at[pl.ds(base, PER_W)], sidx_v)
    pltpu.sync_copy(dstv_h.at[pl.ds(base, PER_W)], didx_v)
    pltpu.sync_copy(z64_h, out_sh.at[pl.ds(s * ROWS_T, ROWS_T)])
    pltpu.sync_copy(z1_h, den_sh.at[pl.ds(s * ROWS_T, ROWS_T)])
    plsc.subcore_barrier()

    def gather_descs(t, b):
        return [
            pltpu.make_async_copy(
                h_h.at[sidx_v.at[pl.ds(t * sb + k * CHUNK, CHUNK)]],
                rows_v.at[pl.ds(b * sb + k * CHUNK, CHUNK)], sem)
            for k in range(SUPER)
        ]

    def scatter_descs(b):
        ds_ = []
        for k in range(SUPER):
            ds_.append(pltpu.make_async_copy(
                exb_v.at[pl.ds(b * (sb + 16) + k * CHUNK, CHUNK)],
                den_sh.at[didx2_v.at[b * SUPER + k]], sem2))
            ds_.append(pltpu.make_async_copy(
                rows_v.at[pl.ds(b * sb + k * CHUNK, CHUNK)],
                out_sh.at[didx2_v.at[b * SUPER + k]], sem2))
        return ds_

    for d in gather_descs(0, 0):
        d.start()

    def super_blk(t, carry):
        b = lax.rem(t, 2)
        for k in range(SUPER):
            @plsc.parallel_loop(0, 8, unroll=8)
            def _(i):
                off = t * sb + k * CHUNK + i * 16
                sv = sidx_v[pl.ds(off, 16)]
                dv = didx_v[pl.ds(off, 16)]
                didx2_v[b * SUPER + k, pl.ds(i * 16, 16)] = dv
                a = plsc.load_gather(asrc_v, [sv]) + plsc.load_gather(adst_v, [dv])
                a = jnp.where(a >= 0.0, a, 0.5 * a)
                exb_v[pl.ds(b * (sb + 16) + k * CHUNK + i * 16, 16)] = jnp.exp(a)
        for d in gather_descs(t, b):
            d.wait()
        @pl.when(t > 0)
        def _():
            for d in scatter_descs(1 - b):
                d.wait()

        @pl.when(t + 1 < ITERS)
        def _():
            for d in gather_descs(t + 1, 1 - b):
                d.start()

        r0 = b * sb
        e0 = b * (sb + 16)

        @plsc.parallel_loop(0, sb, unroll=8)
        def _(r):
            scv = exb_v[pl.ds(e0 + r, 16)][0]
            for q in range(4):
                rows_v[r0 + r, pl.ds(q * 16, 16)] = (
                    rows_v[r0 + r, pl.ds(q * 16, 16)] * scv)

        for d in scatter_descs(b):
            d.start(add=True)
        return carry

    lax.fori_loop(0, ITERS, super_blk, 0)
    for d in scatter_descs((ITERS - 1) % 2):
        d.wait()
    plsc.subcore_barrier()
    rsl = pl.ds(s * ROWS_T, ROWS_T)

    @pl.when(c == 0)
    def _():
        pltpu.sync_copy(out_sh.at[rsl], outp0_h.at[rsl])
        pltpu.sync_copy(den_sh.at[rsl], den0_h.at[rsl])

    @pl.when(c == 1)
    def _():
        pltpu.sync_copy(out_sh.at[rsl], outp1_h.at[rsl])
        pltpu.sync_copy(den_sh.at[rsl], den1_h.at[rsl])


def _sc_call(srcv, dstv, asrc, adst, h, z64, z1):
    mesh = plsc.VectorSubcoreMesh(core_axis_name="c", subcore_axis_name="s")
    return pl.kernel(
        _sc_body,
        out_type=(
            jax.ShapeDtypeStruct((NP_, F), jnp.float32),
            jax.ShapeDtypeStruct((NP_, F), jnp.float32),
            jax.ShapeDtypeStruct((NP_,), jnp.float32),
            jax.ShapeDtypeStruct((NP_,), jnp.float32),
        ),
        mesh=mesh,
        compiler_params=pltpu.CompilerParams(needs_layout_passes=False,
                                             use_tc_tiling_on_sc=False),
        scratch_types=[
            pltpu.VMEM((NP_,), jnp.float32),
            pltpu.VMEM((NP_,), jnp.float32),
            pltpu.VMEM((PER_W,), jnp.int32),
            pltpu.VMEM((PER_W,), jnp.int32),
            pltpu.VMEM((2 * SUPER, CHUNK), jnp.int32),
            pltpu.VMEM((2 * (SUPER * CHUNK + 16),), jnp.float32),
            pltpu.VMEM((2 * SUPER * CHUNK, F), jnp.float32),
            pltpu.VMEM_SHARED((NP_, F), jnp.float32),
            pltpu.VMEM_SHARED((NP_,), jnp.float32),
            pltpu.SemaphoreType.DMA,
            pltpu.SemaphoreType.DMA,
        ],
    )(srcv, dstv, asrc, adst, h, z64, z1)


def _emb_body(o0_ref, o1_ref, d0_ref, d1_ref, bias_ref, h2t_ref,
              emb_ref, xhat_ref):
    o = o0_ref[...] + o1_ref[...]
    dnm = d0_ref[...] + d1_ref[...]
    e = o / (dnm + 1e-16) + bias_ref[...]
    e = jnp.where(e >= 0.0, e, 0.01 * e)
    emb_ref[...] = e
    xhat_ref[...] = lax.dot_general(e, h2t_ref[...], (((1,), (0,)), ((), ())),
                                    preferred_element_type=jnp.float32)


def _tc_emb(outp0, outp1, den0, den1, bias_gat, h2t):
    nb = (N + 511) // 512
    return pl.pallas_call(
        _emb_body,
        grid=(nb,),
        in_specs=[
            pl.BlockSpec((512, F), lambda i: (i, 0)),
            pl.BlockSpec((512, F), lambda i: (i, 0)),
            pl.BlockSpec((512, 1), lambda i: (i, 0)),
            pl.BlockSpec((512, 1), lambda i: (i, 0)),
            pl.BlockSpec((1, F), lambda i: (0, 0)),
            pl.BlockSpec((F, D), lambda i: (0, 0)),
        ],
        out_specs=[
            pl.BlockSpec((512, F), lambda i: (i, 0)),
            pl.BlockSpec((512, D), lambda i: (i, 0)),
        ],
        out_shape=[
            jax.ShapeDtypeStruct((N, F), jnp.float32),
            jax.ShapeDtypeStruct((N, D), jnp.float32),
        ],
    )(outp0, outp1, den0.reshape(NP_, 1), den1.reshape(NP_, 1),
      bias_gat.reshape(1, F), h2t)


def _ahat_body(a_ref, b_ref, o_ref):
    z = lax.dot_general(a_ref[...], b_ref[...], (((1,), (1,)), ((), ())),
                        preferred_element_type=jnp.float32)
    o_ref[...] = 0.5 * jnp.tanh(0.5 * z) + 0.5


def _tc_ahat(emb):
    nbi = (N + 1023) // 1024
    nbj = (N + 4095) // 4096
    return pl.pallas_call(
        _ahat_body,
        grid=(nbi, nbj),
        in_specs=[
            pl.BlockSpec((1024, F), lambda i, j: (i, 0)),
            pl.BlockSpec((4096, F), lambda i, j: (j, 0)),
        ],
        out_specs=pl.BlockSpec((1024, 4096), lambda i, j: (i, j)),
        out_shape=jax.ShapeDtypeStruct((N, N), jnp.float32),
    )(emb, emb)


def kernel(x, edge_index, adj, W_gat, att_src, att_dst, bias_gat, W1, b1, W2, b2):
    e = edge_index.shape[1]
    ei = edge_index.astype(jnp.int32)
    loops = jnp.arange(N, dtype=jnp.int32)
    pad = N + jnp.arange(EP - e - N, dtype=jnp.int32) % (NP_ - N)
    srcv = jnp.concatenate([ei[0], loops, pad])
    dstv = jnp.concatenate([ei[1], loops, pad])

    h, asr, adr, h2t = _tc_pre(x, W_gat, att_src, att_dst, W1, b1, W2, b2)

    z64 = jnp.zeros((ROWS_T, F), jnp.float32)
    z1 = jnp.zeros((ROWS_T,), jnp.float32)
    outp0, outp1, den0, den1 = _sc_call(srcv, dstv, asr.reshape(NP_),
                                        adr.reshape(NP_), h, z64, z1)

    emb, xhat = _tc_emb(outp0, outp1, den0, den1, bias_gat, h2t)
    a_hat = _tc_ahat(emb)
    return (a_hat, xhat)


# speedup vs baseline: 1.2716x; 1.0004x over previous
import jax
import jax.numpy as jnp
from jax import lax
from jax.experimental import pallas as pl
from jax.experimental.pallas import tpu as pltpu
from jax.experimental.pallas import tpu_sc as plsc

N = 10000
D = 128
F = 64
NP_ = 10240
NW = 32
CHUNK = 128
SUPER = 3
PER_W = 5376
ITERS = PER_W // (SUPER * CHUNK)
EP = NW * PER_W
ROWS_T = NP_ // 16


def _pre_body(x_ref, wg_ref, asw_ref, adw_ref, xf_ref, w1_ref, b1_ref,
              w2_ref, b2_ref, h_ref, as_ref, ad_ref, h2t_ref):
    h = lax.dot_general(x_ref[...], wg_ref[...], (((1,), (1,)), ((), ())),
                        preferred_element_type=jnp.float32)
    h_ref[...] = h
    as_ref[...] = lax.dot_general(asw_ref[...], h, (((1,), (1,)), ((), ())),
                                  preferred_element_type=jnp.float32)
    ad_ref[...] = lax.dot_general(adw_ref[...], h, (((1,), (1,)), ((), ())),
                                  preferred_element_type=jnp.float32)

    @pl.when(pl.program_id(0) == 0)
    def _():
        w1x = lax.dot_general(w1_ref[...], xf_ref[...], (((1,), (0,)), ((), ())),
                              preferred_element_type=jnp.float32)
        h1t = jnp.maximum(w1x + b1_ref[...], 0.0)
        h2t_ref[...] = lax.dot_general(w2_ref[...], h1t, (((1,), (0,)), ((), ())),
                                       preferred_element_type=jnp.float32) + b2_ref[...]


def _tc_pre(x, W_gat, att_src, att_dst, W1, b1, W2, b2):
    nb = NP_ // 512
    return pl.pallas_call(
        _pre_body,
        grid=(nb,),
        in_specs=[
            pl.BlockSpec((512, D), lambda i: (i, 0)),
            pl.BlockSpec((F, D), lambda i: (0, 0)),
            pl.BlockSpec((1, F), lambda i: (0, 0)),
            pl.BlockSpec((1, F), lambda i: (0, 0)),
            pl.BlockSpec((N, D), lambda i: (0, 0)),
            pl.BlockSpec((F, N), lambda i: (0, 0)),
            pl.BlockSpec((F, 1), lambda i: (0, 0)),
            pl.BlockSpec((F, F), lambda i: (0, 0)),
            pl.BlockSpec((F, 1), lambda i: (0, 0)),
        ],
        out_specs=[
            pl.BlockSpec((512, F), lambda i: (i, 0)),
            pl.BlockSpec((1, 512), lambda i: (0, i)),
            pl.BlockSpec((1, 512), lambda i: (0, i)),
            pl.BlockSpec((F, D), lambda i: (0, 0)),
        ],
        out_shape=[
            jax.ShapeDtypeStruct((NP_, F), jnp.float32),
            jax.ShapeDtypeStruct((1, NP_), jnp.float32),
            jax.ShapeDtypeStruct((1, NP_), jnp.float32),
            jax.ShapeDtypeStruct((F, D), jnp.float32),
        ],
    )(x, W_gat, att_src.reshape(1, F), att_dst.reshape(1, F),
      x, W1, b1.reshape(F, 1), W2, b2.reshape(F, 1))


def _sc_body(srcv_h, dstv_h, asrc_h, adst_h, h_h, z64_h, z1_h,
             outp0_h, outp1_h, den0_h, den1_h,
             asrc_v, adst_v, sidx_v, didx_v, didx2_v, exb_v, rows_v,
             out_sh, den_sh, sem, sem2):
    c = lax.axis_index("c")
    s = lax.axis_index("s")
    wid = c * 16 + s
    base = wid * PER_W
    sb = SUPER * CHUNK

    pltpu.sync_copy(asrc_h, asrc_v)
    pltpu.sync_copy(adst_h, adst_v)
    pltpu.sync_copy(srcv_h.at[pl.ds(base, PER_W)], sidx_v)
    pltpu.sync_copy(dstv_h.at[pl.ds(base, PER_W)], didx_v)
    pltpu.sync_copy(z64_h, out_sh.at[pl.ds(s * ROWS_T, ROWS_T)])
    pltpu.sync_copy(z1_h, den_sh.at[pl.ds(s * ROWS_T, ROWS_T)])
    plsc.subcore_barrier()

    def gather_descs(t, b):
        return [
            pltpu.make_async_copy(
                h_h.at[sidx_v.at[pl.ds(t * sb + k * CHUNK, CHUNK)]],
                rows_v.at[pl.ds(b * sb + k * CHUNK, CHUNK)], sem)
            for k in range(SUPER)
        ]

    def scatter_descs(b):
        ds_ = []
        for k in range(SUPER):
            ds_.append(pltpu.make_async_copy(
                exb_v.at[pl.ds(b * (sb + 16) + k * CHUNK, CHUNK)],
                den_sh.at[didx2_v.at[b * SUPER + k]], sem2))
            ds_.append(pltpu.make_async_copy(
                rows_v.at[pl.ds(b * sb + k * CHUNK, CHUNK)],
                out_sh.at[didx2_v.at[b * SUPER + k]], sem2))
        return ds_

    for d in gather_descs(0, 0):
        d.start()

    def super_blk(t, carry):
        b = lax.rem(t, 2)
        for k in range(SUPER):
            @plsc.parallel_loop(0, 8, unroll=8)
            def _(i):
                off = t * sb + k * CHUNK + i * 16
                sv = sidx_v[pl.ds(off, 16)]
                dv = didx_v[pl.ds(off, 16)]
                didx2_v[b * SUPER + k, pl.ds(i * 16, 16)] = dv
                a = plsc.load_gather(asrc_v, [sv]) + plsc.load_gather(adst_v, [dv])
                a = jnp.where(a >= 0.0, a, 0.5 * a)
                exb_v[pl.ds(b * (sb + 16) + k * CHUNK + i * 16, 16)] = jnp.exp(a)
        for d in gather_descs(t, b):
            d.wait()
        @pl.when(t > 0)
        def _():
            for d in scatter_descs(1 - b):
                d.wait()

        @pl.when(t + 1 < ITERS)
        def _():
            for d in gather_descs(t + 1, 1 - b):
                d.start()

        r0 = b * sb
        e0 = b * (sb + 16)

        @plsc.parallel_loop(0, sb, unroll=16)
        def _(r):
            scv = exb_v[pl.ds(e0 + r, 16)][0]
            for q in range(4):
                rows_v[r0 + r, pl.ds(q * 16, 16)] = (
                    rows_v[r0 + r, pl.ds(q * 16, 16)] * scv)

        for d in scatter_descs(b):
            d.start(add=True)
        return carry

    lax.fori_loop(0, ITERS, super_blk, 0)
    for d in scatter_descs((ITERS - 1) % 2):
        d.wait()
    plsc.subcore_barrier()
    rsl = pl.ds(s * ROWS_T, ROWS_T)

    @pl.when(c == 0)
    def _():
        pltpu.sync_copy(out_sh.at[rsl], outp0_h.at[rsl])
        pltpu.sync_copy(den_sh.at[rsl], den0_h.at[rsl])

    @pl.when(c == 1)
    def _():
        pltpu.sync_copy(out_sh.at[rsl], outp1_h.at[rsl])
        pltpu.sync_copy(den_sh.at[rsl], den1_h.at[rsl])


def _sc_call(srcv, dstv, asrc, adst, h, z64, z1):
    mesh = plsc.VectorSubcoreMesh(core_axis_name="c", subcore_axis_name="s")
    return pl.kernel(
        _sc_body,
        out_type=(
            jax.ShapeDtypeStruct((NP_, F), jnp.float32),
            jax.ShapeDtypeStruct((NP_, F), jnp.float32),
            jax.ShapeDtypeStruct((NP_,), jnp.float32),
            jax.ShapeDtypeStruct((NP_,), jnp.float32),
        ),
        mesh=mesh,
        compiler_params=pltpu.CompilerParams(needs_layout_passes=False,
                                             use_tc_tiling_on_sc=False),
        scratch_types=[
            pltpu.VMEM((NP_,), jnp.float32),
            pltpu.VMEM((NP_,), jnp.float32),
            pltpu.VMEM((PER_W,), jnp.int32),
            pltpu.VMEM((PER_W,), jnp.int32),
            pltpu.VMEM((2 * SUPER, CHUNK), jnp.int32),
            pltpu.VMEM((2 * (SUPER * CHUNK + 16),), jnp.float32),
            pltpu.VMEM((2 * SUPER * CHUNK, F), jnp.float32),
            pltpu.VMEM_SHARED((NP_, F), jnp.float32),
            pltpu.VMEM_SHARED((NP_,), jnp.float32),
            pltpu.SemaphoreType.DMA,
            pltpu.SemaphoreType.DMA,
        ],
    )(srcv, dstv, asrc, adst, h, z64, z1)


def _emb_body(o0_ref, o1_ref, d0_ref, d1_ref, bias_ref, h2t_ref,
              emb_ref, xhat_ref):
    o = o0_ref[...] + o1_ref[...]
    dnm = d0_ref[...] + d1_ref[...]
    e = o / (dnm + 1e-16) + bias_ref[...]
    e = jnp.where(e >= 0.0, e, 0.01 * e)
    emb_ref[...] = e
    xhat_ref[...] = lax.dot_general(e, h2t_ref[...], (((1,), (0,)), ((), ())),
                                    preferred_element_type=jnp.float32)


def _tc_emb(outp0, outp1, den0, den1, bias_gat, h2t):
    nb = (N + 511) // 512
    return pl.pallas_call(
        _emb_body,
        grid=(nb,),
        in_specs=[
            pl.BlockSpec((512, F), lambda i: (i, 0)),
            pl.BlockSpec((512, F), lambda i: (i, 0)),
            pl.BlockSpec((512, 1), lambda i: (i, 0)),
            pl.BlockSpec((512, 1), lambda i: (i, 0)),
            pl.BlockSpec((1, F), lambda i: (0, 0)),
            pl.BlockSpec((F, D), lambda i: (0, 0)),
        ],
        out_specs=[
            pl.BlockSpec((512, F), lambda i: (i, 0)),
            pl.BlockSpec((512, D), lambda i: (i, 0)),
        ],
        out_shape=[
            jax.ShapeDtypeStruct((N, F), jnp.float32),
            jax.ShapeDtypeStruct((N, D), jnp.float32),
        ],
    )(outp0, outp1, den0.reshape(NP_, 1), den1.reshape(NP_, 1),
      bias_gat.reshape(1, F), h2t)


def _ahat_body(a_ref, b_ref, o_ref):
    z = lax.dot_general(a_ref[...], b_ref[...], (((1,), (1,)), ((), ())),
                        preferred_element_type=jnp.float32)
    o_ref[...] = 0.5 * jnp.tanh(0.5 * z) + 0.5


def _tc_ahat(emb):
    nbi = (N + 1023) // 1024
    nbj = (N + 4095) // 4096
    return pl.pallas_call(
        _ahat_body,
        grid=(nbi, nbj),
        in_specs=[
            pl.BlockSpec((1024, F), lambda i, j: (i, 0)),
            pl.BlockSpec((4096, F), lambda i, j: (j, 0)),
        ],
        out_specs=pl.BlockSpec((1024, 4096), lambda i, j: (i, j)),
        out_shape=jax.ShapeDtypeStruct((N, N), jnp.float32),
    )(emb, emb)


def kernel(x, edge_index, adj, W_gat, att_src, att_dst, bias_gat, W1, b1, W2, b2):
    e = edge_index.shape[1]
    ei = edge_index.astype(jnp.int32)
    loops = jnp.arange(N, dtype=jnp.int32)
    pad = N + jnp.arange(EP - e - N, dtype=jnp.int32) % (NP_ - N)
    srcv = jnp.concatenate([ei[0], loops, pad])
    dstv = jnp.concatenate([ei[1], loops, pad])

    h, asr, adr, h2t = _tc_pre(x, W_gat, att_src, att_dst, W1, b1, W2, b2)

    z64 = jnp.zeros((ROWS_T, F), jnp.float32)
    z1 = jnp.zeros((ROWS_T,), jnp.float32)
    outp0, outp1, den0, den1 = _sc_call(srcv, dstv, asr.reshape(NP_),
                                        adr.reshape(NP_), h, z64, z1)

    emb, xhat = _tc_emb(outp0, outp1, den0, den1, bias_gat, h2t)
    a_hat = _tc_ahat(emb)
    return (a_hat, xhat)


# speedup vs baseline: 1.3113x; 1.0312x over previous
import jax
import jax.numpy as jnp
import numpy as _np
from jax import lax
from jax.experimental import pallas as pl
from jax.experimental.pallas import tpu as pltpu
from jax.experimental.pallas import tpu_sc as plsc

N = 10000
D = 128
F = 64
NP_ = 10240
NW = 32
CHUNK = 128
SUPER = 3
PER_W = 5376
ITERS = PER_W // (SUPER * CHUNK)
EP = NW * PER_W
E_W = 160000 // NW
T_W = PER_W - E_W
ROWS_T = NP_ // 16


def _pre_body(x_ref, wg_ref, asw_ref, adw_ref, xf_ref, w1_ref, b1_ref,
              w2_ref, b2_ref, h_ref, as_ref, ad_ref, h2t_ref):
    h = lax.dot_general(x_ref[...], wg_ref[...], (((1,), (1,)), ((), ())),
                        preferred_element_type=jnp.float32)
    h_ref[...] = h
    as_ref[...] = lax.dot_general(asw_ref[...], h, (((1,), (1,)), ((), ())),
                                  preferred_element_type=jnp.float32)
    ad_ref[...] = lax.dot_general(adw_ref[...], h, (((1,), (1,)), ((), ())),
                                  preferred_element_type=jnp.float32)

    @pl.when(pl.program_id(0) == 0)
    def _():
        w1x = lax.dot_general(w1_ref[...], xf_ref[...], (((1,), (0,)), ((), ())),
                              preferred_element_type=jnp.float32)
        h1t = jnp.maximum(w1x + b1_ref[...], 0.0)
        h2t_ref[...] = lax.dot_general(w2_ref[...], h1t, (((1,), (0,)), ((), ())),
                                       preferred_element_type=jnp.float32) + b2_ref[...]


def _tc_pre(x, W_gat, att_src, att_dst, W1, b1, W2, b2):
    nb = NP_ // 512
    return pl.pallas_call(
        _pre_body,
        grid=(nb,),
        in_specs=[
            pl.BlockSpec((512, D), lambda i: (i, 0)),
            pl.BlockSpec((F, D), lambda i: (0, 0)),
            pl.BlockSpec((1, F), lambda i: (0, 0)),
            pl.BlockSpec((1, F), lambda i: (0, 0)),
            pl.BlockSpec((N, D), lambda i: (0, 0)),
            pl.BlockSpec((F, N), lambda i: (0, 0)),
            pl.BlockSpec((F, 1), lambda i: (0, 0)),
            pl.BlockSpec((F, F), lambda i: (0, 0)),
            pl.BlockSpec((F, 1), lambda i: (0, 0)),
        ],
        out_specs=[
            pl.BlockSpec((512, F), lambda i: (i, 0)),
            pl.BlockSpec((1, 512), lambda i: (0, i)),
            pl.BlockSpec((1, 512), lambda i: (0, i)),
            pl.BlockSpec((F, D), lambda i: (0, 0)),
        ],
        out_shape=[
            jax.ShapeDtypeStruct((NP_, F), jnp.float32),
            jax.ShapeDtypeStruct((1, NP_), jnp.float32),
            jax.ShapeDtypeStruct((1, NP_), jnp.float32),
            jax.ShapeDtypeStruct((F, D), jnp.float32),
        ],
    )(x, W_gat, att_src.reshape(1, F), att_dst.reshape(1, F),
      x, W1, b1.reshape(F, 1), W2, b2.reshape(F, 1))


def _sc_body(s0_h, s1_h, tail_h, asrc_h, adst_h, h_h, z64_h, z1_h,
             outp0_h, outp1_h, den0_h, den1_h,
             asrc_v, adst_v, sidx_v, didx_v, didx2_v, exb_v, rows_v,
             out_sh, den_sh, sem, sem2):
    c = lax.axis_index("c")
    s = lax.axis_index("s")
    wid = c * 16 + s
    sb = SUPER * CHUNK

    pltpu.sync_copy(asrc_h, asrc_v)
    pltpu.sync_copy(adst_h, adst_v)
    pltpu.sync_copy(s0_h.at[pl.ds(wid * E_W, E_W)], sidx_v.at[pl.ds(0, E_W)])
    pltpu.sync_copy(tail_h.at[pl.ds(wid * T_W, T_W)],
                    sidx_v.at[pl.ds(E_W, T_W)])
    pltpu.sync_copy(s1_h.at[pl.ds(wid * E_W, E_W)], didx_v.at[pl.ds(0, E_W)])
    pltpu.sync_copy(tail_h.at[pl.ds(wid * T_W, T_W)],
                    didx_v.at[pl.ds(E_W, T_W)])
    pltpu.sync_copy(z64_h, out_sh.at[pl.ds(s * ROWS_T, ROWS_T)])
    pltpu.sync_copy(z1_h, den_sh.at[pl.ds(s * ROWS_T, ROWS_T)])
    plsc.subcore_barrier()

    def gather_descs(t, b):
        return [
            pltpu.make_async_copy(
                h_h.at[sidx_v.at[pl.ds(t * sb + k * CHUNK, CHUNK)]],
                rows_v.at[pl.ds(b * sb + k * CHUNK, CHUNK)], sem)
            for k in range(SUPER)
        ]

    def scatter_descs(b):
        ds_ = []
        for k in range(SUPER):
            ds_.append(pltpu.make_async_copy(
                exb_v.at[pl.ds(b * (sb + 16) + k * CHUNK, CHUNK)],
                den_sh.at[didx2_v.at[b * SUPER + k]], sem2))
            ds_.append(pltpu.make_async_copy(
                rows_v.at[pl.ds(b * sb + k * CHUNK, CHUNK)],
                out_sh.at[didx2_v.at[b * SUPER + k]], sem2))
        return ds_

    for d in gather_descs(0, 0):
        d.start()

    def super_blk(t, carry):
        b = lax.rem(t, 2)
        for k in range(SUPER):
            @plsc.parallel_loop(0, 8, unroll=8)
            def _(i):
                off = t * sb + k * CHUNK + i * 16
                sv = sidx_v[pl.ds(off, 16)]
                dv = didx_v[pl.ds(off, 16)]
                didx2_v[b * SUPER + k, pl.ds(i * 16, 16)] = dv
                a = plsc.load_gather(asrc_v, [sv]) + plsc.load_gather(adst_v, [dv])
                a = jnp.where(a >= 0.0, a, 0.5 * a)
                exb_v[pl.ds(b * (sb + 16) + k * CHUNK + i * 16, 16)] = jnp.exp(a)
        for d in gather_descs(t, b):
            d.wait()
        @pl.when(t > 0)
        def _():
            for d in scatter_descs(1 - b):
                d.wait()

        @pl.when(t + 1 < ITERS)
        def _():
            for d in gather_descs(t + 1, 1 - b):
                d.start()

        r0 = b * sb
        e0 = b * (sb + 16)

        @plsc.parallel_loop(0, sb, unroll=16)
        def _(r):
            scv = exb_v[pl.ds(e0 + r, 16)][0]
            for q in range(4):
                rows_v[r0 + r, pl.ds(q * 16, 16)] = (
                    rows_v[r0 + r, pl.ds(q * 16, 16)] * scv)

        for d in scatter_descs(b):
            d.start(add=True)
        return carry

    lax.fori_loop(0, ITERS, super_blk, 0)
    for d in scatter_descs((ITERS - 1) % 2):
        d.wait()
    plsc.subcore_barrier()
    rsl = pl.ds(s * ROWS_T, ROWS_T)

    @pl.when(c == 0)
    def _():
        pltpu.sync_copy(out_sh.at[rsl], outp0_h.at[rsl])
        pltpu.sync_copy(den_sh.at[rsl], den0_h.at[rsl])

    @pl.when(c == 1)
    def _():
        pltpu.sync_copy(out_sh.at[rsl], outp1_h.at[rsl])
        pltpu.sync_copy(den_sh.at[rsl], den1_h.at[rsl])


def _sc_call(s0, s1, tail, asrc, adst, h, z64, z1):
    mesh = plsc.VectorSubcoreMesh(core_axis_name="c", subcore_axis_name="s")
    return pl.kernel(
        _sc_body,
        out_type=(
            jax.ShapeDtypeStruct((NP_, F), jnp.float32),
            jax.ShapeDtypeStruct((NP_, F), jnp.float32),
            jax.ShapeDtypeStruct((NP_,), jnp.float32),
            jax.ShapeDtypeStruct((NP_,), jnp.float32),
        ),
        mesh=mesh,
        compiler_params=pltpu.CompilerParams(needs_layout_passes=False,
                                             use_tc_tiling_on_sc=False),
        scratch_types=[
            pltpu.VMEM((NP_,), jnp.float32),
            pltpu.VMEM((NP_,), jnp.float32),
            pltpu.VMEM((PER_W,), jnp.int32),
            pltpu.VMEM((PER_W,), jnp.int32),
            pltpu.VMEM((2 * SUPER, CHUNK), jnp.int32),
            pltpu.VMEM((2 * (SUPER * CHUNK + 16),), jnp.float32),
            pltpu.VMEM((2 * SUPER * CHUNK, F), jnp.float32),
            pltpu.VMEM_SHARED((NP_, F), jnp.float32),
            pltpu.VMEM_SHARED((NP_,), jnp.float32),
            pltpu.SemaphoreType.DMA,
            pltpu.SemaphoreType.DMA,
        ],
    )(s0, s1, tail, asrc, adst, h, z64, z1)


def _emb_body(o0_ref, o1_ref, d0_ref, d1_ref, bias_ref, h2t_ref,
              emb_ref, xhat_ref):
    o = o0_ref[...] + o1_ref[...]
    dnm = d0_ref[...] + d1_ref[...]
    e = o / (dnm + 1e-16) + bias_ref[...]
    e = jnp.where(e >= 0.0, e, 0.01 * e)
    emb_ref[...] = e
    xhat_ref[...] = lax.dot_general(e, h2t_ref[...], (((1,), (0,)), ((), ())),
                                    preferred_element_type=jnp.float32)


def _tc_emb(outp0, outp1, den0, den1, bias_gat, h2t):
    nb = (N + 511) // 512
    return pl.pallas_call(
        _emb_body,
        grid=(nb,),
        in_specs=[
            pl.BlockSpec((512, F), lambda i: (i, 0)),
            pl.BlockSpec((512, F), lambda i: (i, 0)),
            pl.BlockSpec((512, 1), lambda i: (i, 0)),
            pl.BlockSpec((512, 1), lambda i: (i, 0)),
            pl.BlockSpec((1, F), lambda i: (0, 0)),
            pl.BlockSpec((F, D), lambda i: (0, 0)),
        ],
        out_specs=[
            pl.BlockSpec((512, F), lambda i: (i, 0)),
            pl.BlockSpec((512, D), lambda i: (i, 0)),
        ],
        out_shape=[
            jax.ShapeDtypeStruct((N, F), jnp.float32),
            jax.ShapeDtypeStruct((N, D), jnp.float32),
        ],
    )(outp0, outp1, den0.reshape(NP_, 1), den1.reshape(NP_, 1),
      bias_gat.reshape(1, F), h2t)


def _ahat_body(a_ref, b_ref, o_ref):
    z = lax.dot_general(a_ref[...], b_ref[...], (((1,), (1,)), ((), ())),
                        preferred_element_type=jnp.float32)
    o_ref[...] = 0.5 * jnp.tanh(0.5 * z) + 0.5


def _tc_ahat(emb):
    nbi = (N + 1023) // 1024
    nbj = (N + 4095) // 4096
    return pl.pallas_call(
        _ahat_body,
        grid=(nbi, nbj),
        in_specs=[
            pl.BlockSpec((1024, F), lambda i, j: (i, 0)),
            pl.BlockSpec((4096, F), lambda i, j: (j, 0)),
        ],
        out_specs=pl.BlockSpec((1024, 4096), lambda i, j: (i, j)),
        out_shape=jax.ShapeDtypeStruct((N, N), jnp.float32),
    )(emb, emb)


def kernel(x, edge_index, adj, W_gat, att_src, att_dst, bias_gat, W1, b1, W2, b2):
    ei = edge_index.astype(jnp.int32)
    tail = jnp.asarray(_np.concatenate([
        _np.arange(N, dtype=_np.int32),
        N + _np.arange(NW * T_W - N, dtype=_np.int32) % (NP_ - N)]))

    h, asr, adr, h2t = _tc_pre(x, W_gat, att_src, att_dst, W1, b1, W2, b2)

    z64 = jnp.zeros((ROWS_T, F), jnp.float32)
    z1 = jnp.zeros((ROWS_T,), jnp.float32)
    outp0, outp1, den0, den1 = _sc_call(ei[0], ei[1], tail, asr.reshape(NP_),
                                        adr.reshape(NP_), h, z64, z1)

    emb, xhat = _tc_emb(outp0, outp1, den0, den1, bias_gat, h2t)
    a_hat = _tc_ahat(emb)
    return (a_hat, xhat)


# speedup vs baseline: 1.3311x; 1.0151x over previous
import jax
import jax.numpy as jnp
import numpy as _np
from jax import lax
from jax.experimental import pallas as pl
from jax.experimental.pallas import tpu as pltpu
from jax.experimental.pallas import tpu_sc as plsc

N = 10000
D = 128
F = 64
NP_ = 10240
NW = 32
CHUNK = 128
SUPER = 3
PER_W = 5376
ITERS = PER_W // (SUPER * CHUNK)
EP = NW * PER_W
E_W = 160000 // NW
T_W = PER_W - E_W
ROWS_T = NP_ // 16


def _pre_body(x_ref, wg_ref, asw_ref, adw_ref, xf_ref, w1_ref, b1_ref,
              w2_ref, b2_ref, h_ref, as_ref, ad_ref, h2t_ref):
    h = lax.dot_general(x_ref[...], wg_ref[...], (((1,), (1,)), ((), ())),
                        preferred_element_type=jnp.float32)
    h_ref[...] = h
    as_ref[...] = lax.dot_general(asw_ref[...], h, (((1,), (1,)), ((), ())),
                                  preferred_element_type=jnp.float32)
    ad_ref[...] = lax.dot_general(adw_ref[...], h, (((1,), (1,)), ((), ())),
                                  preferred_element_type=jnp.float32)

    @pl.when(pl.program_id(0) == 0)
    def _():
        w1x = lax.dot_general(w1_ref[...], xf_ref[...], (((1,), (0,)), ((), ())),
                              preferred_element_type=jnp.float32)
        h1t = jnp.maximum(w1x + b1_ref[...], 0.0)
        h2t_ref[...] = lax.dot_general(w2_ref[...], h1t, (((1,), (0,)), ((), ())),
                                       preferred_element_type=jnp.float32) + b2_ref[...]


def _tc_pre(x, W_gat, att_src, att_dst, W1, b1, W2, b2):
    nb = NP_ // 512
    return pl.pallas_call(
        _pre_body,
        grid=(nb,),
        in_specs=[
            pl.BlockSpec((512, D), lambda i: (i, 0)),
            pl.BlockSpec((F, D), lambda i: (0, 0)),
            pl.BlockSpec((1, F), lambda i: (0, 0)),
            pl.BlockSpec((1, F), lambda i: (0, 0)),
            pl.BlockSpec((N, D), lambda i: (0, 0)),
            pl.BlockSpec((F, N), lambda i: (0, 0)),
            pl.BlockSpec((F, 1), lambda i: (0, 0)),
            pl.BlockSpec((F, F), lambda i: (0, 0)),
            pl.BlockSpec((F, 1), lambda i: (0, 0)),
        ],
        out_specs=[
            pl.BlockSpec((512, F), lambda i: (i, 0)),
            pl.BlockSpec((1, 512), lambda i: (0, i)),
            pl.BlockSpec((1, 512), lambda i: (0, i)),
            pl.BlockSpec((F, D), lambda i: (0, 0)),
        ],
        out_shape=[
            jax.ShapeDtypeStruct((NP_, F), jnp.float32),
            jax.ShapeDtypeStruct((1, NP_), jnp.float32),
            jax.ShapeDtypeStruct((1, NP_), jnp.float32),
            jax.ShapeDtypeStruct((F, D), jnp.float32),
        ],
    )(x, W_gat, att_src.reshape(1, F), att_dst.reshape(1, F),
      x, W1, b1.reshape(F, 1), W2, b2.reshape(F, 1))


def _sc_body(s0_h, s1_h, tail_h, asrc_h, adst_h, h_h, z64_h, z1_h,
             outp0_h, outp1_h, den0_h, den1_h,
             asrc_v, adst_v, sidx_v, didx_v, didx2_v, exb_v, rows_v,
             out_sh, den_sh, sem, sem2):
    c = lax.axis_index("c")
    s = lax.axis_index("s")
    wid = c * 16 + s
    sb = SUPER * CHUNK

    pltpu.sync_copy(asrc_h, asrc_v)
    pltpu.sync_copy(adst_h, adst_v)
    pltpu.sync_copy(s0_h.at[pl.ds(wid * E_W, E_W)], sidx_v.at[pl.ds(0, E_W)])
    pltpu.sync_copy(tail_h.at[pl.ds(wid * T_W, T_W)],
                    sidx_v.at[pl.ds(E_W, T_W)])
    pltpu.sync_copy(s1_h.at[pl.ds(wid * E_W, E_W)], didx_v.at[pl.ds(0, E_W)])
    pltpu.sync_copy(tail_h.at[pl.ds(wid * T_W, T_W)],
                    didx_v.at[pl.ds(E_W, T_W)])
    pltpu.sync_copy(z64_h, out_sh.at[pl.ds(s * ROWS_T, ROWS_T)])
    pltpu.sync_copy(z1_h, den_sh.at[pl.ds(s * ROWS_T, ROWS_T)])
    plsc.subcore_barrier()

    def gather_descs(t, b):
        return [
            pltpu.make_async_copy(
                h_h.at[sidx_v.at[pl.ds(t * sb + k * CHUNK, CHUNK)]],
                rows_v.at[pl.ds(b * sb + k * CHUNK, CHUNK)], sem)
            for k in range(SUPER)
        ]

    def scatter_descs(b):
        ds_ = []
        for k in range(SUPER):
            ds_.append(pltpu.make_async_copy(
                exb_v.at[pl.ds(b * (sb + 16) + k * CHUNK, CHUNK)],
                den_sh.at[didx2_v.at[b * SUPER + k]], sem2))
            ds_.append(pltpu.make_async_copy(
                rows_v.at[pl.ds(b * sb + k * CHUNK, CHUNK)],
                out_sh.at[didx2_v.at[b * SUPER + k]], sem2))
        return ds_

    for d in gather_descs(0, 0):
        d.start()

    def super_blk(t, carry):
        b = lax.rem(t, 2)
        for k in range(SUPER):
            @plsc.parallel_loop(0, 8, unroll=8)
            def _(i):
                off = t * sb + k * CHUNK + i * 16
                sv = sidx_v[pl.ds(off, 16)]
                dv = didx_v[pl.ds(off, 16)]
                didx2_v[b * SUPER + k, pl.ds(i * 16, 16)] = dv
                a = plsc.load_gather(asrc_v, [sv]) + plsc.load_gather(adst_v, [dv])
                a = jnp.where(a >= 0.0, a, 0.5 * a)
                exb_v[pl.ds(b * (sb + 16) + k * CHUNK + i * 16, 16)] = jnp.exp(a)
        for d in gather_descs(t, b):
            d.wait()
        @pl.when(t > 0)
        def _():
            for d in scatter_descs(1 - b):
                d.wait()

        @pl.when(t + 1 < ITERS)
        def _():
            for d in gather_descs(t + 1, 1 - b):
                d.start()

        r0 = b * sb
        e0 = b * (sb + 16)

        @plsc.parallel_loop(0, sb, unroll=16)
        def _(r):
            scv = exb_v[pl.ds(e0 + r, 16)][0]
            for q in range(4):
                rows_v[r0 + r, pl.ds(q * 16, 16)] = (
                    rows_v[r0 + r, pl.ds(q * 16, 16)] * scv)

        for d in scatter_descs(b):
            d.start(add=True)
        return carry

    lax.fori_loop(0, ITERS, super_blk, 0)
    for d in scatter_descs((ITERS - 1) % 2):
        d.wait()
    plsc.subcore_barrier()
    rsl = pl.ds(s * ROWS_T, ROWS_T)

    @pl.when(c == 0)
    def _():
        pltpu.sync_copy(out_sh.at[rsl], outp0_h.at[rsl])
        pltpu.sync_copy(den_sh.at[rsl], den0_h.at[rsl])

    @pl.when(c == 1)
    def _():
        pltpu.sync_copy(out_sh.at[rsl], outp1_h.at[rsl])
        pltpu.sync_copy(den_sh.at[rsl], den1_h.at[rsl])


def _sc_call(s0, s1, tail, asrc, adst, h, z64, z1):
    mesh = plsc.VectorSubcoreMesh(core_axis_name="c", subcore_axis_name="s")
    return pl.kernel(
        _sc_body,
        out_type=(
            jax.ShapeDtypeStruct((NP_, F), jnp.float32),
            jax.ShapeDtypeStruct((NP_, F), jnp.float32),
            jax.ShapeDtypeStruct((NP_,), jnp.float32),
            jax.ShapeDtypeStruct((NP_,), jnp.float32),
        ),
        mesh=mesh,
        compiler_params=pltpu.CompilerParams(needs_layout_passes=False,
                                             use_tc_tiling_on_sc=False),
        scratch_types=[
            pltpu.VMEM((NP_,), jnp.float32),
            pltpu.VMEM((NP_,), jnp.float32),
            pltpu.VMEM((PER_W,), jnp.int32),
            pltpu.VMEM((PER_W,), jnp.int32),
            pltpu.VMEM((2 * SUPER, CHUNK), jnp.int32),
            pltpu.VMEM((2 * (SUPER * CHUNK + 16),), jnp.float32),
            pltpu.VMEM((2 * SUPER * CHUNK, F), jnp.float32),
            pltpu.VMEM_SHARED((NP_, F), jnp.float32),
            pltpu.VMEM_SHARED((NP_,), jnp.float32),
            pltpu.SemaphoreType.DMA,
            pltpu.SemaphoreType.DMA,
        ],
    )(s0, s1, tail, asrc, adst, h, z64, z1)


def _emb_body(o0_ref, o1_ref, d0_ref, d1_ref, bias_ref, h2t_ref,
              emb_ref, xhat_ref):
    o = o0_ref[...] + o1_ref[...]
    dnm = d0_ref[...] + d1_ref[...]
    e = o / (dnm + 1e-16) + bias_ref[...]
    e = jnp.where(e >= 0.0, e, 0.01 * e)
    emb_ref[...] = e
    xhat_ref[...] = lax.dot_general(e, h2t_ref[...], (((1,), (0,)), ((), ())),
                                    preferred_element_type=jnp.float32)


def _tc_emb(outp0, outp1, den0, den1, bias_gat, h2t):
    nb = (N + 511) // 512
    return pl.pallas_call(
        _emb_body,
        grid=(nb,),
        in_specs=[
            pl.BlockSpec((512, F), lambda i: (i, 0)),
            pl.BlockSpec((512, F), lambda i: (i, 0)),
            pl.BlockSpec((512, 1), lambda i: (i, 0)),
            pl.BlockSpec((512, 1), lambda i: (i, 0)),
            pl.BlockSpec((1, F), lambda i: (0, 0)),
            pl.BlockSpec((F, D), lambda i: (0, 0)),
        ],
        out_specs=[
            pl.BlockSpec((512, F), lambda i: (i, 0)),
            pl.BlockSpec((512, D), lambda i: (i, 0)),
        ],
        out_shape=[
            jax.ShapeDtypeStruct((N, F), jnp.float32),
            jax.ShapeDtypeStruct((N, D), jnp.float32),
        ],
    )(outp0, outp1, den0.reshape(NP_, 1), den1.reshape(NP_, 1),
      bias_gat.reshape(1, F), h2t)


def _ahat_body(a_ref, b_ref, o_ref):
    z = lax.dot_general(a_ref[...], b_ref[...], (((1,), (1,)), ((), ())),
                        preferred_element_type=jnp.float32)
    o_ref[...] = 0.5 * jnp.tanh(0.5 * z) + 0.5


def _tc_ahat(emb):
    nbi = (N + 1023) // 1024
    nbj = (N + 5119) // 5120
    return pl.pallas_call(
        _ahat_body,
        grid=(nbi, nbj),
        in_specs=[
            pl.BlockSpec((1024, F), lambda i, j: (i, 0)),
            pl.BlockSpec((5120, F), lambda i, j: (j, 0)),
        ],
        out_specs=pl.BlockSpec((1024, 5120), lambda i, j: (i, j)),
        out_shape=jax.ShapeDtypeStruct((N, N), jnp.float32),
    )(emb, emb)


def kernel(x, edge_index, adj, W_gat, att_src, att_dst, bias_gat, W1, b1, W2, b2):
    ei = edge_index.astype(jnp.int32)
    tail = jnp.asarray(_np.concatenate([
        _np.arange(N, dtype=_np.int32),
        N + _np.arange(NW * T_W - N, dtype=_np.int32) % (NP_ - N)]))

    h, asr, adr, h2t = _tc_pre(x, W_gat, att_src, att_dst, W1, b1, W2, b2)

    z64 = jnp.zeros((ROWS_T, F), jnp.float32)
    z1 = jnp.zeros((ROWS_T,), jnp.float32)
    outp0, outp1, den0, den1 = _sc_call(ei[0], ei[1], tail, asr.reshape(NP_),
                                        adr.reshape(NP_), h, z64, z1)

    emb, xhat = _tc_emb(outp0, outp1, den0, den1, bias_gat, h2t)
    a_hat = _tc_ahat(emb)
    return (a_hat, xhat)


# speedup vs baseline: 1.3467x; 1.0117x over previous
import jax
import jax.numpy as jnp
import numpy as _np
from jax import lax
from jax.experimental import pallas as pl
from jax.experimental.pallas import tpu as pltpu
from jax.experimental.pallas import tpu_sc as plsc

N = 10000
D = 128
F = 64
NP_ = 10240
NW = 32
CHUNK = 128
SUPER = 3
PER_W = 5376
ITERS = PER_W // (SUPER * CHUNK)
EP = NW * PER_W
E_W = 160000 // NW
T_W = PER_W - E_W
ROWS_T = NP_ // 16


def _pre_body(x_ref, wg_ref, asw_ref, adw_ref, xf_ref, w1_ref, b1_ref,
              w2_ref, b2_ref, h_ref, as_ref, ad_ref, h2t_ref):
    h = lax.dot_general(x_ref[...], wg_ref[...], (((1,), (1,)), ((), ())),
                        preferred_element_type=jnp.float32)
    h_ref[...] = h
    as_ref[...] = lax.dot_general(asw_ref[...], h, (((1,), (1,)), ((), ())),
                                  preferred_element_type=jnp.float32)
    ad_ref[...] = lax.dot_general(adw_ref[...], h, (((1,), (1,)), ((), ())),
                                  preferred_element_type=jnp.float32)

    @pl.when(pl.program_id(0) == 0)
    def _():
        w1x = lax.dot_general(w1_ref[...], xf_ref[...], (((1,), (0,)), ((), ())),
                              preferred_element_type=jnp.float32)
        h1t = jnp.maximum(w1x + b1_ref[...], 0.0)
        h2t_ref[...] = lax.dot_general(w2_ref[...], h1t, (((1,), (0,)), ((), ())),
                                       preferred_element_type=jnp.float32) + b2_ref[...]


def _tc_pre(x, W_gat, att_src, att_dst, W1, b1, W2, b2):
    nb = NP_ // 512
    return pl.pallas_call(
        _pre_body,
        grid=(nb,),
        in_specs=[
            pl.BlockSpec((512, D), lambda i: (i, 0)),
            pl.BlockSpec((F, D), lambda i: (0, 0)),
            pl.BlockSpec((1, F), lambda i: (0, 0)),
            pl.BlockSpec((1, F), lambda i: (0, 0)),
            pl.BlockSpec((N, D), lambda i: (0, 0)),
            pl.BlockSpec((F, N), lambda i: (0, 0)),
            pl.BlockSpec((F, 1), lambda i: (0, 0)),
            pl.BlockSpec((F, F), lambda i: (0, 0)),
            pl.BlockSpec((F, 1), lambda i: (0, 0)),
        ],
        out_specs=[
            pl.BlockSpec((512, F), lambda i: (i, 0)),
            pl.BlockSpec((1, 512), lambda i: (0, i)),
            pl.BlockSpec((1, 512), lambda i: (0, i)),
            pl.BlockSpec((F, D), lambda i: (0, 0)),
        ],
        out_shape=[
            jax.ShapeDtypeStruct((NP_, F), jnp.float32),
            jax.ShapeDtypeStruct((1, NP_), jnp.float32),
            jax.ShapeDtypeStruct((1, NP_), jnp.float32),
            jax.ShapeDtypeStruct((F, D), jnp.float32),
        ],
    )(x, W_gat, att_src.reshape(1, F), att_dst.reshape(1, F),
      x, W1, b1.reshape(F, 1), W2, b2.reshape(F, 1))


def _sc_body(s0_h, s1_h, tail_h, asrc_h, adst_h, h_h, z64_h, z1_h,
             outp0_h, outp1_h, den0_h, den1_h,
             asrc_v, adst_v, sidx_v, didx_v, didx2_v, exb_v, rows_v,
             out_sh, den_sh, sem, sem2):
    c = lax.axis_index("c")
    s = lax.axis_index("s")
    wid = c * 16 + s
    sb = SUPER * CHUNK

    pltpu.sync_copy(asrc_h, asrc_v)
    pltpu.sync_copy(adst_h, adst_v)
    pltpu.sync_copy(s0_h.at[pl.ds(wid * E_W, E_W)], sidx_v.at[pl.ds(0, E_W)])
    pltpu.sync_copy(tail_h.at[pl.ds(wid * T_W, T_W)],
                    sidx_v.at[pl.ds(E_W, T_W)])
    pltpu.sync_copy(s1_h.at[pl.ds(wid * E_W, E_W)], didx_v.at[pl.ds(0, E_W)])
    pltpu.sync_copy(tail_h.at[pl.ds(wid * T_W, T_W)],
                    didx_v.at[pl.ds(E_W, T_W)])
    pltpu.sync_copy(z64_h, out_sh.at[pl.ds(s * ROWS_T, ROWS_T)])
    pltpu.sync_copy(z1_h, den_sh.at[pl.ds(s * ROWS_T, ROWS_T)])
    plsc.subcore_barrier()

    def gather_descs(t, b):
        return [
            pltpu.make_async_copy(
                h_h.at[sidx_v.at[pl.ds(t * sb + k * CHUNK, CHUNK)]],
                rows_v.at[pl.ds(b * sb + k * CHUNK, CHUNK)], sem)
            for k in range(SUPER)
        ]

    def scatter_descs(b):
        ds_ = []
        for k in range(SUPER):
            ds_.append(pltpu.make_async_copy(
                exb_v.at[pl.ds(b * (sb + 16) + k * CHUNK, CHUNK)],
                den_sh.at[didx2_v.at[b * SUPER + k]], sem2))
            ds_.append(pltpu.make_async_copy(
                rows_v.at[pl.ds(b * sb + k * CHUNK, CHUNK)],
                out_sh.at[didx2_v.at[b * SUPER + k]], sem2))
        return ds_

    for d in gather_descs(0, 0):
        d.start()

    def super_blk(t, carry):
        b = lax.rem(t, 2)
        for k in range(SUPER):
            @plsc.parallel_loop(0, 8, unroll=8)
            def _(i):
                off = t * sb + k * CHUNK + i * 16
                sv = sidx_v[pl.ds(off, 16)]
                dv = didx_v[pl.ds(off, 16)]
                didx2_v[b * SUPER + k, pl.ds(i * 16, 16)] = dv
                a = plsc.load_gather(asrc_v, [sv]) + plsc.load_gather(adst_v, [dv])
                a = jnp.where(a >= 0.0, a, 0.5 * a)
                exb_v[pl.ds(b * (sb + 16) + k * CHUNK + i * 16, 16)] = jnp.exp(a)
        for d in gather_descs(t, b):
            d.wait()
        @pl.when(t > 0)
        def _():
            for d in scatter_descs(1 - b):
                d.wait()

        @pl.when(t + 1 < ITERS)
        def _():
            for d in gather_descs(t + 1, 1 - b):
                d.start()

        r0 = b * sb
        e0 = b * (sb + 16)

        @plsc.parallel_loop(0, sb, unroll=16)
        def _(r):
            scv = exb_v[pl.ds(e0 + r, 16)][0]
            for q in range(4):
                rows_v[r0 + r, pl.ds(q * 16, 16)] = (
                    rows_v[r0 + r, pl.ds(q * 16, 16)] * scv)

        for d in scatter_descs(b):
            d.start(add=True)
        return carry

    lax.fori_loop(0, ITERS, super_blk, 0)
    for d in scatter_descs((ITERS - 1) % 2):
        d.wait()
    plsc.subcore_barrier()
    rsl = pl.ds(s * ROWS_T, ROWS_T)

    @pl.when(c == 0)
    def _():
        pltpu.sync_copy(out_sh.at[rsl], outp0_h.at[rsl])
        pltpu.sync_copy(den_sh.at[rsl], den0_h.at[rsl])

    @pl.when(c == 1)
    def _():
        pltpu.sync_copy(out_sh.at[rsl], outp1_h.at[rsl])
        pltpu.sync_copy(den_sh.at[rsl], den1_h.at[rsl])


def _sc_call(s0, s1, tail, asrc, adst, h, z64, z1):
    mesh = plsc.VectorSubcoreMesh(core_axis_name="c", subcore_axis_name="s")
    return pl.kernel(
        _sc_body,
        out_type=(
            jax.ShapeDtypeStruct((NP_, F), jnp.float32),
            jax.ShapeDtypeStruct((NP_, F), jnp.float32),
            jax.ShapeDtypeStruct((NP_,), jnp.float32),
            jax.ShapeDtypeStruct((NP_,), jnp.float32),
        ),
        mesh=mesh,
        compiler_params=pltpu.CompilerParams(needs_layout_passes=False,
                                             use_tc_tiling_on_sc=False),
        scratch_types=[
            pltpu.VMEM((NP_,), jnp.float32),
            pltpu.VMEM((NP_,), jnp.float32),
            pltpu.VMEM((PER_W,), jnp.int32),
            pltpu.VMEM((PER_W,), jnp.int32),
            pltpu.VMEM((2 * SUPER, CHUNK), jnp.int32),
            pltpu.VMEM((2 * (SUPER * CHUNK + 16),), jnp.float32),
            pltpu.VMEM((2 * SUPER * CHUNK, F), jnp.float32),
            pltpu.VMEM_SHARED((NP_, F), jnp.float32),
            pltpu.VMEM_SHARED((NP_,), jnp.float32),
            pltpu.SemaphoreType.DMA,
            pltpu.SemaphoreType.DMA,
        ],
    )(s0, s1, tail, asrc, adst, h, z64, z1)


def _emb_body(o0_ref, o1_ref, d0_ref, d1_ref, bias_ref, h2t_ref,
              emb_ref, xhat_ref):
    o = o0_ref[...] + o1_ref[...]
    dnm = d0_ref[...] + d1_ref[...]
    e = o / (dnm + 1e-16) + bias_ref[...]
    e = jnp.where(e >= 0.0, e, 0.01 * e)
    emb_ref[...] = e
    xhat_ref[...] = lax.dot_general(e, h2t_ref[...], (((1,), (0,)), ((), ())),
                                    preferred_element_type=jnp.float32)


def _tc_emb(outp0, outp1, den0, den1, bias_gat, h2t):
    nb = (N + 511) // 512
    return pl.pallas_call(
        _emb_body,
        grid=(nb,),
        in_specs=[
            pl.BlockSpec((512, F), lambda i: (i, 0)),
            pl.BlockSpec((512, F), lambda i: (i, 0)),
            pl.BlockSpec((512, 1), lambda i: (i, 0)),
            pl.BlockSpec((512, 1), lambda i: (i, 0)),
            pl.BlockSpec((1, F), lambda i: (0, 0)),
            pl.BlockSpec((F, D), lambda i: (0, 0)),
        ],
        out_specs=[
            pl.BlockSpec((512, F), lambda i: (i, 0)),
            pl.BlockSpec((512, D), lambda i: (i, 0)),
        ],
        out_shape=[
            jax.ShapeDtypeStruct((N, F), jnp.float32),
            jax.ShapeDtypeStruct((N, D), jnp.float32),
        ],
    )(outp0, outp1, den0.reshape(NP_, 1), den1.reshape(NP_, 1),
      bias_gat.reshape(1, F), h2t)


def _ahat_body(a_ref, b_ref, o_ref):
    z = lax.dot_general(a_ref[...], b_ref[...], (((1,), (1,)), ((), ())),
                        preferred_element_type=jnp.float32)
    o_ref[...] = 0.5 * jnp.tanh(0.5 * z) + 0.5


def _tc_ahat(emb):
    nbi = (N + 1279) // 1280
    nbj = (N + 5119) // 5120
    return pl.pallas_call(
        _ahat_body,
        grid=(nbi, nbj),
        in_specs=[
            pl.BlockSpec((1280, F), lambda i, j: (i, 0)),
            pl.BlockSpec((5120, F), lambda i, j: (j, 0)),
        ],
        out_specs=pl.BlockSpec((1280, 5120), lambda i, j: (i, j)),
        out_shape=jax.ShapeDtypeStruct((N, N), jnp.float32),
    )(emb, emb)


def kernel(x, edge_index, adj, W_gat, att_src, att_dst, bias_gat, W1, b1, W2, b2):
    ei = edge_index.astype(jnp.int32)
    tail = jnp.asarray(_np.concatenate([
        _np.arange(N, dtype=_np.int32),
        N + _np.arange(NW * T_W - N, dtype=_np.int32) % (NP_ - N)]))

    h, asr, adr, h2t = _tc_pre(x, W_gat, att_src, att_dst, W1, b1, W2, b2)

    z64 = jnp.zeros((ROWS_T, F), jnp.float32)
    z1 = jnp.zeros((ROWS_T,), jnp.float32)
    outp0, outp1, den0, den1 = _sc_call(ei[0], ei[1], tail, asr.reshape(NP_),
                                        adr.reshape(NP_), h, z64, z1)

    emb, xhat = _tc_emb(outp0, outp1, den0, den1, bias_gat, h2t)
    a_hat = _tc_ahat(emb)
    return (a_hat, xhat)
